# Initial kernel scaffold; baseline (speedup 1.0000x reference)
import jax
import jax.numpy as jnp
from jax.experimental import pallas as pl


def kernel(velocity, tracer, node_x, node_y, cell_area_at_node, dt, node_at_link_head, node_at_link_tail, links_at_node, upwind_real_idx):
    raise NotImplementedError("write your pallas kernel here")


# speedup vs baseline: 150.3866x; 150.3866x over previous
import functools

import jax
import jax.numpy as jnp
from jax import lax
from jax.experimental import pallas as pl
from jax.experimental.pallas import tpu as pltpu
from jax.experimental.pallas import tpu_sc as plsc

N_NODES = 100_000
N_LINKS = 1_600_000
LPN = 16

NC = 2
NS = 16
NW = NC * NS

LW = N_LINKS // NW
CL = 2_000
NLC = LW // CL

CN = 800
NCH = N_NODES // CN

_MESH = plsc.VectorSubcoreMesh(
    core_axis_name="c", subcore_axis_name="s", num_cores=NC, num_subcores=NS)

_f32 = jnp.float32
_i32 = jnp.int32


def _wid():
    return lax.axis_index("s") * NC + lax.axis_index("c")


def _hbm_to_spmem(hbm_ref, sp_ref, vtmp, base, nchunks, csz=None):
    if csz is None:
        csz = vtmp.shape[0]

    def body(i, carry):
        o = pl.multiple_of(base + i * csz, 8)
        sl = pl.ds(o, csz)
        pltpu.sync_copy(hbm_ref.at[sl], vtmp.at[pl.ds(0, csz)])
        pltpu.sync_copy(vtmp.at[pl.ds(0, csz)], sp_ref.at[sl])
        return carry

    lax.fori_loop(0, nchunks, body, 0)


def _stage_a_body(nx, ny, tr, hidx, tidx, vel,
                  gx, gy, cen, dif, gox, goy,
                  xs, ys, ts,
                  hidx_v, tidx_v, vel_v,
                  hx_v, hy_v, htr_v, tx_v, ty_v, ttr_v,
                  gx_v, gy_v, cen_v, dif_v, gox_v, goy_v, stg_v, sem):
    s = lax.axis_index("s")

    @pl.when(s < 4)
    def _stage_tables():
        base = s * 25_000
        _hbm_to_spmem(nx, xs, stg_v, base, 1)
        _hbm_to_spmem(ny, ys, stg_v, base, 1)
        _hbm_to_spmem(tr, ts, stg_v, base, 1)

    plsc.subcore_barrier()
    base = _wid() * LW

    def chunk(i, carry):
        off = pl.multiple_of(base + i * CL, 8)
        sl = pl.ds(off, CL)
        pltpu.sync_copy(hidx.at[sl], hidx_v)
        pltpu.sync_copy(tidx.at[sl], tidx_v)
        pltpu.sync_copy(vel.at[sl], vel_v)
        cps = (pltpu.async_copy(xs.at[hidx_v], hx_v, sem),
               pltpu.async_copy(ys.at[hidx_v], hy_v, sem),
               pltpu.async_copy(ts.at[hidx_v], htr_v, sem),
               pltpu.async_copy(xs.at[tidx_v], tx_v, sem),
               pltpu.async_copy(ys.at[tidx_v], ty_v, sem),
               pltpu.async_copy(ts.at[tidx_v], ttr_v, sem))
        for cp in cps:
            cp.wait()

        def vec(j, carry2):
            vs = pl.ds(j * 16, 16)
            hx = hx_v[vs]; hy = hy_v[vs]; htr = htr_v[vs]
            tx = tx_v[vs]; ty = ty_v[vs]; ttr = ttr_v[vs]
            v = vel_v[vs]
            dx = hx - tx
            dy = hy - ty
            l2 = jnp.maximum(dx * dx + dy * dy, 1e-18)
            dtr = (htr - ttr) / l2
            gx_v[vs] = dtr * dx
            gy_v[vs] = dtr * dy
            vpos = v >= 0.0
            cen_v[vs] = jnp.where(vpos, ttr, htr)
            dif_v[vs] = jnp.where(vpos, htr - ttr, ttr - htr)
            gox_v[vs] = jnp.where(vpos, 2.0 * hx - tx, 2.0 * tx - hx)
            goy_v[vs] = jnp.where(vpos, 2.0 * hy - ty, 2.0 * ty - hy)
            return carry2

        lax.fori_loop(0, CL // 16, vec, 0)
        pltpu.sync_copy(gx_v, gx.at[sl])
        pltpu.sync_copy(gy_v, gy.at[sl])
        pltpu.sync_copy(cen_v, cen.at[sl])
        pltpu.sync_copy(dif_v, dif.at[sl])
        pltpu.sync_copy(gox_v, gox.at[sl])
        pltpu.sync_copy(goy_v, goy.at[sl])
        return carry

    lax.fori_loop(0, NLC, chunk, 0)


_stage_a = pl.kernel(
    _stage_a_body,
    out_type=tuple(jax.ShapeDtypeStruct((N_LINKS,), _f32) for _ in range(6)),
    mesh=_MESH,
    compiler_params=pltpu.CompilerParams(needs_layout_passes=False),
    scratch_types=(
        [pltpu.VMEM_SHARED((N_NODES,), _f32) for _ in range(3)]
        + [pltpu.VMEM((CL,), _i32) for _ in range(2)]
        + [pltpu.VMEM((CL,), _f32) for _ in range(13)]
        + [pltpu.VMEM((25_000,), _f32)]
        + [pltpu.SemaphoreType.DMA]
    ),
)


def _stage_b_body(gx, gy, lan, gxn, gyn,
                  fsp, lan_v, g_v, out_v, sem):
    c = lax.axis_index("c")
    s = lax.axis_index("s")

    def run(field, out):
        _hbm_to_spmem(field, fsp, g_v, s * (N_LINKS // NS), 10, csz=10_000)
        plsc.subcore_barrier()
        iota16 = lax.iota(_i32, 16) * 16

        def chunk(i, carry):
            ch = s + i * NS

            @pl.when(ch < NCH)
            def _():
                off = pl.multiple_of(ch * CN * LPN, 8)
                pltpu.sync_copy(lan.at[pl.ds(off, CN * LPN)], lan_v)
                pltpu.async_copy(fsp.at[lan_v], g_v, sem).wait()

                def red(ii, carry2):
                    b = ii * (16 * LPN)
                    acc = jnp.zeros((16,), _f32)
                    for k in range(LPN):
                        acc = acc + plsc.load_gather(g_v, [iota16 + (b + k)])
                    out_v[pl.ds(ii * 16, 16)] = acc * (1.0 / LPN)
                    return carry2

                lax.fori_loop(0, CN // 16, red, 0)
                pltpu.sync_copy(out_v, out.at[pl.ds(pl.multiple_of(ch * CN, 8), CN)])
            return carry

        lax.fori_loop(0, (NCH + NS - 1) // NS, chunk, 0)

    @pl.when(c == 0)
    def _():
        run(gx, gxn)

    @pl.when(c == 1)
    def _():
        run(gy, gyn)


_stage_b = pl.kernel(
    _stage_b_body,
    out_type=tuple(jax.ShapeDtypeStruct((N_NODES,), _f32) for _ in range(2)),
    mesh=_MESH,
    compiler_params=pltpu.CompilerParams(needs_layout_passes=False),
    scratch_types=(
        pltpu.VMEM_SHARED((N_LINKS,), _f32),
        pltpu.VMEM((CN * LPN,), _i32),
        pltpu.VMEM((CN * LPN,), _f32),
        pltpu.VMEM((CN,), _f32),
        pltpu.SemaphoreType.DMA,
    ),
)


def _stage_c_body(nx, ny, tr, gxn, gyn, uidx, vel, cen, dif, gox, goy,
                  flux,
                  xs, ys, ts, gxs, gys,
                  uidx_v, vel_v, cen_v, dif_v, gox_v, goy_v,
                  ux_v, uy_v, utr_v, ugx_v, ugy_v, flux_v, stg_v, sem):
    s = lax.axis_index("s")

    @pl.when(s < 4)
    def _stage_tables():
        base = s * 25_000
        _hbm_to_spmem(nx, xs, stg_v, base, 1)
        _hbm_to_spmem(ny, ys, stg_v, base, 1)
        _hbm_to_spmem(tr, ts, stg_v, base, 1)
        _hbm_to_spmem(gxn, gxs, stg_v, base, 1)
        _hbm_to_spmem(gyn, gys, stg_v, base, 1)

    plsc.subcore_barrier()
    base = _wid() * LW

    def chunk(i, carry):
        off = pl.multiple_of(base + i * CL, 8)
        sl = pl.ds(off, CL)
        pltpu.sync_copy(uidx.at[sl], uidx_v)
        pltpu.sync_copy(vel.at[sl], vel_v)
        pltpu.sync_copy(cen.at[sl], cen_v)
        pltpu.sync_copy(dif.at[sl], dif_v)
        pltpu.sync_copy(gox.at[sl], gox_v)
        pltpu.sync_copy(goy.at[sl], goy_v)
        cps = (pltpu.async_copy(xs.at[uidx_v], ux_v, sem),
               pltpu.async_copy(ys.at[uidx_v], uy_v, sem),
               pltpu.async_copy(ts.at[uidx_v], utr_v, sem),
               pltpu.async_copy(gxs.at[uidx_v], ugx_v, sem),
               pltpu.async_copy(gys.at[uidx_v], ugy_v, sem))
        for cp in cps:
            cp.wait()

        def vec(j, carry2):
            vs = pl.ds(j * 16, 16)
            up = utr_v[vs] + ((ux_v[vs] - gox_v[vs]) * ugx_v[vs]
                              + (uy_v[vs] - goy_v[vs]) * ugy_v[vs])
            ce = cen_v[vs]
            di = dif_v[vs]
            nz = di != 0.0
            den = jnp.where(nz, di, 1.0)
            r = jnp.where(nz, (ce - up) / den, 0.0)
            ar = jnp.abs(r)
            phi = (r + ar) / (1.0 + ar)
            flux_v[vs] = vel_v[vs] * (ce + 0.5 * phi * di)
            return carry2

        lax.fori_loop(0, CL // 16, vec, 0)
        pltpu.sync_copy(flux_v, flux.at[sl])
        return carry

    lax.fori_loop(0, NLC, chunk, 0)


_stage_c = pl.kernel(
    _stage_c_body,
    out_type=jax.ShapeDtypeStruct((N_LINKS,), _f32),
    mesh=_MESH,
    compiler_params=pltpu.CompilerParams(needs_layout_passes=False),
    scratch_types=(
        [pltpu.VMEM_SHARED((N_NODES,), _f32) for _ in range(5)]
        + [pltpu.VMEM((CL,), _i32)]
        + [pltpu.VMEM((CL,), _f32) for _ in range(11)]
        + [pltpu.VMEM((25_000,), _f32)]
        + [pltpu.SemaphoreType.DMA]
    ),
)


def _stage_d_body(flux, lan, tr, area, dt16, out,
                  fsp, lan_v, g_v, tr_v, ar_v, dt_v, out_v, sem):
    s = lax.axis_index("s")
    w = _wid()

    _hbm_to_spmem(flux, fsp, g_v, s * (N_LINKS // NS), 10, csz=10_000)
    pltpu.sync_copy(dt16, dt_v)
    plsc.subcore_barrier()
    dtv = dt_v[...]
    iota16 = lax.iota(_i32, 16) * 16

    def chunk(i, carry):
        ch = w + i * NW

        @pl.when(ch < NCH)
        def _():
            noff = pl.multiple_of(ch * CN, 8)
            off = pl.multiple_of(ch * CN * LPN, 8)
            pltpu.sync_copy(lan.at[pl.ds(off, CN * LPN)], lan_v)
            pltpu.sync_copy(tr.at[pl.ds(noff, CN)], tr_v)
            pltpu.sync_copy(area.at[pl.ds(noff, CN)], ar_v)
            pltpu.async_copy(fsp.at[lan_v], g_v, sem).wait()

            def red(ii, carry2):
                b = ii * (16 * LPN)
                acc = jnp.zeros((16,), _f32)
                for k in range(LPN):
                    acc = acc + plsc.load_gather(g_v, [iota16 + (b + k)])
                vs = pl.ds(ii * 16, 16)
                a = ar_v[vs]
                nz = a != 0.0
                asafe = jnp.where(nz, a, 1.0)
                div = jnp.where(nz, acc / asafe, 0.0)
                out_v[vs] = tr_v[vs] + dtv * div
                return carry2

            lax.fori_loop(0, CN // 16, red, 0)
            pltpu.sync_copy(out_v, out.at[pl.ds(noff, CN)])
        return carry

    lax.fori_loop(0, (NCH + NW - 1) // NW, chunk, 0)


_stage_d = pl.kernel(
    _stage_d_body,
    out_type=jax.ShapeDtypeStruct((N_NODES,), _f32),
    mesh=_MESH,
    compiler_params=pltpu.CompilerParams(needs_layout_passes=False),
    scratch_types=(
        pltpu.VMEM_SHARED((N_LINKS,), _f32),
        pltpu.VMEM((CN * LPN,), _i32),
        pltpu.VMEM((CN * LPN,), _f32),
        pltpu.VMEM((CN,), _f32),
        pltpu.VMEM((CN,), _f32),
        pltpu.VMEM((16,), _f32),
        pltpu.VMEM((CN,), _f32),
        pltpu.SemaphoreType.DMA,
    ),
)


def kernel(velocity, tracer, node_x, node_y, cell_area_at_node, dt,
           node_at_link_head, node_at_link_tail, links_at_node,
           upwind_real_idx):
    hidx = node_at_link_head.astype(_i32)
    tidx = node_at_link_tail.astype(_i32)
    uidx = upwind_real_idx.astype(_i32)
    lan_flat = links_at_node.astype(_i32).reshape(-1)
    dt16 = jnp.broadcast_to(dt.astype(_f32), (16,))

    gx, gy, cen, dif, gox, goy = _stage_a(
        node_x, node_y, tracer, hidx, tidx, velocity)
    gxn, gyn = _stage_b(gx, gy, lan_flat)
    flux = _stage_c(node_x, node_y, tracer, gxn, gyn, uidx, velocity,
                    cen, dif, gox, goy)
    return _stage_d(flux, lan_flat, tracer, cell_area_at_node, dt16)


# speedup vs baseline: 159.5376x; 1.0609x over previous
import functools

import jax
import jax.numpy as jnp
from jax import lax
from jax.experimental import pallas as pl
from jax.experimental.pallas import tpu as pltpu
from jax.experimental.pallas import tpu_sc as plsc

N_NODES = 100_000
N_LINKS = 1_600_000
LPN = 16

NC = 2
NS = 16
NW = NC * NS

LW = N_LINKS // NW
CL = 2_000
NLC = LW // CL

CN = 800
NCH = N_NODES // CN

_MESH = plsc.VectorSubcoreMesh(
    core_axis_name="c", subcore_axis_name="s", num_cores=NC, num_subcores=NS)

_f32 = jnp.float32
_i32 = jnp.int32


def _wid():
    return lax.axis_index("s") * NC + lax.axis_index("c")


def _hbm_to_spmem(hbm_ref, sp_ref, vtmp, base, nchunks, csz=None):
    if csz is None:
        csz = vtmp.shape[0]

    def body(i, carry):
        o = pl.multiple_of(base + i * csz, 8)
        sl = pl.ds(o, csz)
        pltpu.sync_copy(hbm_ref.at[sl], vtmp.at[pl.ds(0, csz)])
        pltpu.sync_copy(vtmp.at[pl.ds(0, csz)], sp_ref.at[sl])
        return carry

    lax.fori_loop(0, nchunks, body, 0)


def _stage_a_body(nx, ny, tr, hidx, tidx, vel,
                  gx, gy, cen, dif, gox, goy,
                  xs, ys, ts,
                  hidx_v, tidx_v, vel_v,
                  hx_v, hy_v, htr_v, tx_v, ty_v, ttr_v,
                  gx_v, gy_v, cen_v, dif_v, gox_v, goy_v, stg_v, sem):
    s = lax.axis_index("s")
    base = _wid() * LW

    @pl.when(s < 4)
    def _stage_tables():
        base_t = s * 25_000
        _hbm_to_spmem(nx, xs, stg_v, base_t, 1)
        _hbm_to_spmem(ny, ys, stg_v, base_t, 1)
        _hbm_to_spmem(tr, ts, stg_v, base_t, 1)

    plsc.subcore_barrier()

    def chunk(i, carry):
        off = pl.multiple_of(base + i * CL, 8)
        sl = pl.ds(off, CL)
        pltpu.sync_copy(hidx.at[sl], hidx_v)
        pltpu.sync_copy(tidx.at[sl], tidx_v)
        pltpu.sync_copy(vel.at[sl], vel_v)
        cps = (pltpu.async_copy(xs.at[hidx_v], hx_v, sem),
               pltpu.async_copy(ys.at[hidx_v], hy_v, sem),
               pltpu.async_copy(ts.at[hidx_v], htr_v, sem),
               pltpu.async_copy(xs.at[tidx_v], tx_v, sem),
               pltpu.async_copy(ys.at[tidx_v], ty_v, sem),
               pltpu.async_copy(ts.at[tidx_v], ttr_v, sem))
        for cp in cps:
            cp.wait()

        def vec(j, carry2):
            vs = pl.ds(j * 16, 16)
            hx = hx_v[vs]; hy = hy_v[vs]; htr = htr_v[vs]
            tx = tx_v[vs]; ty = ty_v[vs]; ttr = ttr_v[vs]
            v = vel_v[vs]
            dx = hx - tx
            dy = hy - ty
            l2 = jnp.maximum(dx * dx + dy * dy, 1e-18)
            dtr = (htr - ttr) / l2
            gx_v[vs] = dtr * dx
            gy_v[vs] = dtr * dy
            vpos = v >= 0.0
            cen_v[vs] = jnp.where(vpos, ttr, htr)
            dif_v[vs] = jnp.where(vpos, htr - ttr, ttr - htr)
            gox_v[vs] = jnp.where(vpos, 2.0 * hx - tx, 2.0 * tx - hx)
            goy_v[vs] = jnp.where(vpos, 2.0 * hy - ty, 2.0 * ty - hy)
            return carry2

        lax.fori_loop(0, CL // 16, vec, 0)
        pltpu.sync_copy(gx_v, gx.at[sl])
        pltpu.sync_copy(gy_v, gy.at[sl])
        pltpu.sync_copy(cen_v, cen.at[sl])
        pltpu.sync_copy(dif_v, dif.at[sl])
        pltpu.sync_copy(gox_v, gox.at[sl])
        pltpu.sync_copy(goy_v, goy.at[sl])
        return carry

    lax.fori_loop(0, NLC, chunk, 0)


_stage_a = pl.kernel(
    _stage_a_body,
    out_type=tuple(jax.ShapeDtypeStruct((N_LINKS,), _f32) for _ in range(6)),
    mesh=_MESH,
    compiler_params=pltpu.CompilerParams(needs_layout_passes=False),
    scratch_types=(
        [pltpu.VMEM_SHARED((N_NODES,), _f32) for _ in range(3)]
        + [pltpu.VMEM((CL,), _i32) for _ in range(2)]
        + [pltpu.VMEM((CL,), _f32) for _ in range(13)]
        + [pltpu.VMEM((25_000,), _f32)]
        + [pltpu.SemaphoreType.DMA]
    ),
)


def _stage_b_body(gx, gy, lan, gxn, gyn,
                  fsp, lan_v, g_v, out_v, sem):
    c = lax.axis_index("c")
    s = lax.axis_index("s")

    def run(field, out):
        _hbm_to_spmem(field, fsp, g_v, s * (N_LINKS // NS), 10, csz=10_000)
        plsc.subcore_barrier()
        iota16 = lax.iota(_i32, 16) * 16

        def chunk(i, carry):
            ch = s + i * NS

            @pl.when(ch < NCH)
            def _():
                off = pl.multiple_of(ch * CN * LPN, 8)
                pltpu.sync_copy(lan.at[pl.ds(off, CN * LPN)], lan_v)
                pltpu.async_copy(fsp.at[lan_v], g_v, sem).wait()

                def red(ii, carry2):
                    b = ii * (16 * LPN)
                    acc = jnp.zeros((16,), _f32)
                    for k in range(LPN):
                        acc = acc + plsc.load_gather(g_v, [iota16 + (b + k)])
                    out_v[pl.ds(ii * 16, 16)] = acc * (1.0 / LPN)
                    return carry2

                lax.fori_loop(0, CN // 16, red, 0)
                pltpu.sync_copy(out_v, out.at[pl.ds(pl.multiple_of(ch * CN, 8), CN)])
            return carry

        lax.fori_loop(0, (NCH + NS - 1) // NS, chunk, 0)

    @pl.when(c == 0)
    def _():
        run(gx, gxn)

    @pl.when(c == 1)
    def _():
        run(gy, gyn)


_stage_b = pl.kernel(
    _stage_b_body,
    out_type=tuple(jax.ShapeDtypeStruct((N_NODES,), _f32) for _ in range(2)),
    mesh=_MESH,
    compiler_params=pltpu.CompilerParams(needs_layout_passes=False),
    scratch_types=(
        pltpu.VMEM_SHARED((N_LINKS,), _f32),
        pltpu.VMEM((CN * LPN,), _i32),
        pltpu.VMEM((CN * LPN,), _f32),
        pltpu.VMEM((CN,), _f32),
        pltpu.SemaphoreType.DMA,
    ),
)


def _stage_c_body(nx, ny, tr, gxn, gyn, uidx, vel, cen, dif, gox, goy,
                  flux,
                  ps, gxs, gys,
                  uidx_v, vel_v, cen_v, dif_v, gox_v, goy_v,
                  up_v, ugx_v, ugy_v, flux_v,
                  sx_v, sy_v, st_v, sgx_v, sgy_v, sem):
    s = lax.axis_index("s")

    @pl.when(s < 10)
    def _stage_tables():
        o = pl.multiple_of(s * 10_000, 8)
        slt = pl.ds(o, 10_000)
        pltpu.sync_copy(nx.at[slt], sx_v)
        pltpu.sync_copy(ny.at[slt], sy_v)
        pltpu.sync_copy(tr.at[slt], st_v)
        pltpu.sync_copy(gxn.at[slt], sgx_v)
        pltpu.sync_copy(gyn.at[slt], sgy_v)

        def pbody(j, carry):
            vs = pl.ds(j * 16, 16)
            st_v[vs] = (st_v[vs] + sx_v[vs] * sgx_v[vs]
                        + sy_v[vs] * sgy_v[vs])
            return carry

        lax.fori_loop(0, 10_000 // 16, pbody, 0)
        pltpu.sync_copy(st_v, ps.at[slt])
        pltpu.sync_copy(sgx_v, gxs.at[slt])
        pltpu.sync_copy(sgy_v, gys.at[slt])

    plsc.subcore_barrier()
    base = _wid() * LW

    def chunk(i, carry):
        off = pl.multiple_of(base + i * CL, 8)
        sl = pl.ds(off, CL)
        pltpu.sync_copy(uidx.at[sl], uidx_v)
        pltpu.sync_copy(vel.at[sl], vel_v)
        pltpu.sync_copy(cen.at[sl], cen_v)
        pltpu.sync_copy(dif.at[sl], dif_v)
        pltpu.sync_copy(gox.at[sl], gox_v)
        pltpu.sync_copy(goy.at[sl], goy_v)
        cps = (pltpu.async_copy(ps.at[uidx_v], up_v, sem),
               pltpu.async_copy(gxs.at[uidx_v], ugx_v, sem),
               pltpu.async_copy(gys.at[uidx_v], ugy_v, sem))
        for cp in cps:
            cp.wait()

        def vec(j, carry2):
            vs = pl.ds(j * 16, 16)
            up = (up_v[vs] - gox_v[vs] * ugx_v[vs]
                  - goy_v[vs] * ugy_v[vs])
            ce = cen_v[vs]
            di = dif_v[vs]
            nz = di != 0.0
            den = jnp.where(nz, di, 1.0)
            r = jnp.where(nz, (ce - up) / den, 0.0)
            ar = jnp.abs(r)
            phi = (r + ar) / (1.0 + ar)
            flux_v[vs] = vel_v[vs] * (ce + 0.5 * phi * di)
            return carry2

        lax.fori_loop(0, CL // 16, vec, 0)
        pltpu.sync_copy(flux_v, flux.at[sl])
        return carry

    lax.fori_loop(0, NLC, chunk, 0)


_stage_c = pl.kernel(
    _stage_c_body,
    out_type=jax.ShapeDtypeStruct((N_LINKS,), _f32),
    mesh=_MESH,
    compiler_params=pltpu.CompilerParams(needs_layout_passes=False),
    scratch_types=(
        [pltpu.VMEM_SHARED((N_NODES,), _f32) for _ in range(3)]
        + [pltpu.VMEM((CL,), _i32)]
        + [pltpu.VMEM((CL,), _f32) for _ in range(9)]
        + [pltpu.VMEM((10_000,), _f32) for _ in range(5)]
        + [pltpu.SemaphoreType.DMA]
    ),
)


def _stage_d_body(flux, lan, tr, area, dt16, out,
                  fsp, lan_v, g_v, tr_v, ar_v, dt_v, out_v, sem):
    s = lax.axis_index("s")
    w = _wid()

    _hbm_to_spmem(flux, fsp, g_v, s * (N_LINKS // NS), 10, csz=10_000)
    pltpu.sync_copy(dt16, dt_v)
    plsc.subcore_barrier()
    dtv = dt_v[...]
    iota16 = lax.iota(_i32, 16) * 16

    def chunk(i, carry):
        ch = w + i * NW

        @pl.when(ch < NCH)
        def _():
            noff = pl.multiple_of(ch * CN, 8)
            off = pl.multiple_of(ch * CN * LPN, 8)
            pltpu.sync_copy(lan.at[pl.ds(off, CN * LPN)], lan_v)
            pltpu.sync_copy(tr.at[pl.ds(noff, CN)], tr_v)
            pltpu.sync_copy(area.at[pl.ds(noff, CN)], ar_v)
            pltpu.async_copy(fsp.at[lan_v], g_v, sem).wait()

            def red(ii, carry2):
                b = ii * (16 * LPN)
                acc = jnp.zeros((16,), _f32)
                for k in range(LPN):
                    acc = acc + plsc.load_gather(g_v, [iota16 + (b + k)])
                vs = pl.ds(ii * 16, 16)
                a = ar_v[vs]
                nz = a != 0.0
                asafe = jnp.where(nz, a, 1.0)
                div = jnp.where(nz, acc / asafe, 0.0)
                out_v[vs] = tr_v[vs] + dtv * div
                return carry2

            lax.fori_loop(0, CN // 16, red, 0)
            pltpu.sync_copy(out_v, out.at[pl.ds(noff, CN)])
        return carry

    lax.fori_loop(0, (NCH + NW - 1) // NW, chunk, 0)


_stage_d = pl.kernel(
    _stage_d_body,
    out_type=jax.ShapeDtypeStruct((N_NODES,), _f32),
    mesh=_MESH,
    compiler_params=pltpu.CompilerParams(needs_layout_passes=False),
    scratch_types=(
        pltpu.VMEM_SHARED((N_LINKS,), _f32),
        pltpu.VMEM((CN * LPN,), _i32),
        pltpu.VMEM((CN * LPN,), _f32),
        pltpu.VMEM((CN,), _f32),
        pltpu.VMEM((CN,), _f32),
        pltpu.VMEM((16,), _f32),
        pltpu.VMEM((CN,), _f32),
        pltpu.SemaphoreType.DMA,
    ),
)


def kernel(velocity, tracer, node_x, node_y, cell_area_at_node, dt,
           node_at_link_head, node_at_link_tail, links_at_node,
           upwind_real_idx):
    hidx = node_at_link_head.astype(_i32)
    tidx = node_at_link_tail.astype(_i32)
    uidx = upwind_real_idx.astype(_i32)
    lan_flat = links_at_node.astype(_i32).reshape(-1)
    dt16 = jnp.broadcast_to(dt.astype(_f32), (16,))

    gx, gy, cen, dif, gox, goy = _stage_a(
        node_x, node_y, tracer, hidx, tidx, velocity)
    gxn, gyn = _stage_b(gx, gy, lan_flat)
    flux = _stage_c(node_x, node_y, tracer, gxn, gyn, uidx, velocity,
                    cen, dif, gox, goy)
    return _stage_d(flux, lan_flat, tracer, cell_area_at_node, dt16)


# speedup vs baseline: 208.4927x; 1.3069x over previous
import functools

import jax
import jax.numpy as jnp
from jax import lax
from jax.experimental import pallas as pl
from jax.experimental.pallas import tpu as pltpu
from jax.experimental.pallas import tpu_sc as plsc

N_NODES = 100_000
N_LINKS = 1_600_000
LPN = 16

NC = 2
NS = 16
NW = NC * NS

LW = N_LINKS // NW
CL = 2_000
NLC = LW // CL

CN = 800
NCH = N_NODES // CN

_MESH = plsc.VectorSubcoreMesh(
    core_axis_name="c", subcore_axis_name="s", num_cores=NC, num_subcores=NS)

_f32 = jnp.float32
_i32 = jnp.int32


def _wid():
    return lax.axis_index("s") * NC + lax.axis_index("c")


def _hbm_to_spmem(hbm_ref, sp_ref, vtmp, base, nchunks, csz=None):
    if csz is None:
        csz = vtmp.shape[0]

    def body(i, carry):
        o = pl.multiple_of(base + i * csz, 8)
        sl = pl.ds(o, csz)
        pltpu.sync_copy(hbm_ref.at[sl], vtmp.at[pl.ds(0, csz)])
        pltpu.sync_copy(vtmp.at[pl.ds(0, csz)], sp_ref.at[sl])
        return carry

    lax.fori_loop(0, nchunks, body, 0)


def _stage_a_body(nx, ny, tr, hidx, tidx, vel,
                  gx, gy, cen, dif, gox, goy,
                  xs, ys, ts,
                  hidx_a, tidx_a, vel_a, hx_a, hy_a, htr_a, tx_a, ty_a, ttr_a,
                  hidx_b, tidx_b, vel_b, hx_b, hy_b, htr_b, tx_b, ty_b, ttr_b,
                  gx_v, gy_v, cen_v, dif_v, gox_v, goy_v, stg_v,
                  sem_a, sem_b):
    s = lax.axis_index("s")
    base = _wid() * LW

    @pl.when(s < 4)
    def _stage_tables():
        base_t = s * 25_000
        _hbm_to_spmem(nx, xs, stg_v, base_t, 1)
        _hbm_to_spmem(ny, ys, stg_v, base_t, 1)
        _hbm_to_spmem(tr, ts, stg_v, base_t, 1)

    plsc.subcore_barrier()

    seta = (hidx_a, tidx_a, vel_a, hx_a, hy_a, htr_a, tx_a, ty_a, ttr_a,
            sem_a)
    setb = (hidx_b, tidx_b, vel_b, hx_b, hy_b, htr_b, tx_b, ty_b, ttr_b,
            sem_b)

    def load_fire(ci, st):
        hidx_v, tidx_v, vel_v, hx_v, hy_v, htr_v, tx_v, ty_v, ttr_v, sem = st
        sl = pl.ds(pl.multiple_of(base + ci * CL, 8), CL)
        pltpu.sync_copy(hidx.at[sl], hidx_v)
        pltpu.sync_copy(tidx.at[sl], tidx_v)
        pltpu.sync_copy(vel.at[sl], vel_v)
        pltpu.async_copy(xs.at[hidx_v], hx_v, sem)
        pltpu.async_copy(ys.at[hidx_v], hy_v, sem)
        pltpu.async_copy(ts.at[hidx_v], htr_v, sem)
        pltpu.async_copy(xs.at[tidx_v], tx_v, sem)
        pltpu.async_copy(ys.at[tidx_v], ty_v, sem)
        pltpu.async_copy(ts.at[tidx_v], ttr_v, sem)

    def drain(st):
        sem = st[-1]
        dummy = nx.at[pl.ds(0, CL)]
        for dst in st[3:9]:
            pltpu.make_async_copy(dummy, dst, sem).wait()

    def compute_store(ci, st):
        _, _, vel_v, hx_v, hy_v, htr_v, tx_v, ty_v, ttr_v, _ = st
        sl = pl.ds(pl.multiple_of(base + ci * CL, 8), CL)

        def vec(j, carry2):
            vs = pl.ds(j * 16, 16)
            hx = hx_v[vs]; hy = hy_v[vs]; htr = htr_v[vs]
            tx = tx_v[vs]; ty = ty_v[vs]; ttr = ttr_v[vs]
            v = vel_v[vs]
            dx = hx - tx
            dy = hy - ty
            l2 = jnp.maximum(dx * dx + dy * dy, 1e-18)
            dtr = (htr - ttr) / l2
            gx_v[vs] = dtr * dx
            gy_v[vs] = dtr * dy
            vpos = v >= 0.0
            cen_v[vs] = jnp.where(vpos, ttr, htr)
            dif_v[vs] = jnp.where(vpos, htr - ttr, ttr - htr)
            gox_v[vs] = jnp.where(vpos, 2.0 * hx - tx, 2.0 * tx - hx)
            goy_v[vs] = jnp.where(vpos, 2.0 * hy - ty, 2.0 * ty - hy)
            return carry2

        lax.fori_loop(0, CL // 16, vec, 0)
        pltpu.sync_copy(gx_v, gx.at[sl])
        pltpu.sync_copy(gy_v, gy.at[sl])
        pltpu.sync_copy(cen_v, cen.at[sl])
        pltpu.sync_copy(dif_v, dif.at[sl])
        pltpu.sync_copy(gox_v, gox.at[sl])
        pltpu.sync_copy(goy_v, goy.at[sl])

    load_fire(0, seta)

    def pipe(i, carry):
        c0 = i * 2
        load_fire(c0 + 1, setb)
        drain(seta)
        compute_store(c0, seta)
        load_fire(c0 + 2, seta)
        drain(setb)
        compute_store(c0 + 1, setb)
        return carry

    lax.fori_loop(0, NLC // 2, pipe, 0)
    drain(seta)
    compute_store(NLC - 1, seta)


_stage_a = pl.kernel(
    _stage_a_body,
    out_type=tuple(jax.ShapeDtypeStruct((N_LINKS,), _f32) for _ in range(6)),
    mesh=_MESH,
    compiler_params=pltpu.CompilerParams(needs_layout_passes=False),
    scratch_types=(
        [pltpu.VMEM_SHARED((N_NODES,), _f32) for _ in range(3)]
        + 2 * ([pltpu.VMEM((CL,), _i32) for _ in range(2)]
               + [pltpu.VMEM((CL,), _f32) for _ in range(7)])
        + [pltpu.VMEM((CL,), _f32) for _ in range(6)]
        + [pltpu.VMEM((25_000,), _f32)]
        + [pltpu.SemaphoreType.DMA, pltpu.SemaphoreType.DMA]
    ),
)


def _stage_b_body(gx, gy, lan, gxn, gyn,
                  fsp, lan_v, g_v, out_v, sem):
    c = lax.axis_index("c")
    s = lax.axis_index("s")

    def run(field, out):
        _hbm_to_spmem(field, fsp, g_v, s * (N_LINKS // NS), 10, csz=10_000)
        plsc.subcore_barrier()
        iota16 = lax.iota(_i32, 16) * 16

        def chunk(i, carry):
            ch = s + i * NS

            @pl.when(ch < NCH)
            def _():
                off = pl.multiple_of(ch * CN * LPN, 8)
                pltpu.sync_copy(lan.at[pl.ds(off, CN * LPN)], lan_v)
                pltpu.async_copy(fsp.at[lan_v], g_v, sem).wait()

                def red(ii, carry2):
                    b = ii * (16 * LPN)
                    acc = jnp.zeros((16,), _f32)
                    for k in range(LPN):
                        acc = acc + plsc.load_gather(g_v, [iota16 + (b + k)])
                    out_v[pl.ds(ii * 16, 16)] = acc * (1.0 / LPN)
                    return carry2

                lax.fori_loop(0, CN // 16, red, 0)
                pltpu.sync_copy(out_v, out.at[pl.ds(pl.multiple_of(ch * CN, 8), CN)])
            return carry

        lax.fori_loop(0, (NCH + NS - 1) // NS, chunk, 0)

    @pl.when(c == 0)
    def _():
        run(gx, gxn)

    @pl.when(c == 1)
    def _():
        run(gy, gyn)


_stage_b = pl.kernel(
    _stage_b_body,
    out_type=tuple(jax.ShapeDtypeStruct((N_NODES,), _f32) for _ in range(2)),
    mesh=_MESH,
    compiler_params=pltpu.CompilerParams(needs_layout_passes=False),
    scratch_types=(
        pltpu.VMEM_SHARED((N_LINKS,), _f32),
        pltpu.VMEM((CN * LPN,), _i32),
        pltpu.VMEM((CN * LPN,), _f32),
        pltpu.VMEM((CN,), _f32),
        pltpu.SemaphoreType.DMA,
    ),
)


def _stage_c_body(nx, ny, tr, gxn, gyn, uidx, vel, cen, dif, gox, goy,
                  flux,
                  ps, gxs, gys,
                  uidx_a, vel_a, cen_a, dif_a, gox_a, goy_a,
                  up_a, ugx_a, ugy_a,
                  uidx_b, vel_b, cen_b, dif_b, gox_b, goy_b,
                  up_b, ugx_b, ugy_b,
                  flux_v,
                  sx_v, sy_v, st_v, sgx_v, sgy_v, sem_a, sem_b):
    s = lax.axis_index("s")

    @pl.when(s < 10)
    def _stage_tables():
        o = pl.multiple_of(s * 10_000, 8)
        slt = pl.ds(o, 10_000)
        pltpu.sync_copy(nx.at[slt], sx_v)
        pltpu.sync_copy(ny.at[slt], sy_v)
        pltpu.sync_copy(tr.at[slt], st_v)
        pltpu.sync_copy(gxn.at[slt], sgx_v)
        pltpu.sync_copy(gyn.at[slt], sgy_v)

        def pbody(j, carry):
            vs = pl.ds(j * 16, 16)
            st_v[vs] = (st_v[vs] + sx_v[vs] * sgx_v[vs]
                        + sy_v[vs] * sgy_v[vs])
            return carry

        lax.fori_loop(0, 10_000 // 16, pbody, 0)
        pltpu.sync_copy(st_v, ps.at[slt])
        pltpu.sync_copy(sgx_v, gxs.at[slt])
        pltpu.sync_copy(sgy_v, gys.at[slt])

    plsc.subcore_barrier()
    base = _wid() * LW

    seta = (uidx_a, vel_a, cen_a, dif_a, gox_a, goy_a, up_a, ugx_a, ugy_a,
            sem_a)
    setb = (uidx_b, vel_b, cen_b, dif_b, gox_b, goy_b, up_b, ugx_b, ugy_b,
            sem_b)

    def load_fire(ci, st):
        uidx_v, vel_v, cen_v, dif_v, gox_v, goy_v, up_v, ugx_v, ugy_v, sem = st
        sl = pl.ds(pl.multiple_of(base + ci * CL, 8), CL)
        pltpu.sync_copy(uidx.at[sl], uidx_v)
        pltpu.sync_copy(vel.at[sl], vel_v)
        pltpu.sync_copy(cen.at[sl], cen_v)
        pltpu.sync_copy(dif.at[sl], dif_v)
        pltpu.sync_copy(gox.at[sl], gox_v)
        pltpu.sync_copy(goy.at[sl], goy_v)
        pltpu.async_copy(ps.at[uidx_v], up_v, sem)
        pltpu.async_copy(gxs.at[uidx_v], ugx_v, sem)
        pltpu.async_copy(gys.at[uidx_v], ugy_v, sem)

    def drain(st):
        sem = st[-1]
        dummy = nx.at[pl.ds(0, CL)]
        for dst in st[6:9]:
            pltpu.make_async_copy(dummy, dst, sem).wait()

    def compute_store(ci, st):
        _, vel_v, cen_v, dif_v, gox_v, goy_v, up_v, ugx_v, ugy_v, _ = st
        sl = pl.ds(pl.multiple_of(base + ci * CL, 8), CL)

        def vec(j, carry2):
            vs = pl.ds(j * 16, 16)
            up = (up_v[vs] - gox_v[vs] * ugx_v[vs]
                  - goy_v[vs] * ugy_v[vs])
            ce = cen_v[vs]
            di = dif_v[vs]
            nz = di != 0.0
            den = jnp.where(nz, di, 1.0)
            r = jnp.where(nz, (ce - up) / den, 0.0)
            ar = jnp.abs(r)
            phi = (r + ar) / (1.0 + ar)
            flux_v[vs] = vel_v[vs] * (ce + 0.5 * phi * di)
            return carry2

        lax.fori_loop(0, CL // 16, vec, 0)
        pltpu.sync_copy(flux_v, flux.at[sl])

    load_fire(0, seta)

    def pipe(i, carry):
        c0 = i * 2
        load_fire(c0 + 1, setb)
        drain(seta)
        compute_store(c0, seta)
        load_fire(c0 + 2, seta)
        drain(setb)
        compute_store(c0 + 1, setb)
        return carry

    lax.fori_loop(0, NLC // 2, pipe, 0)
    drain(seta)
    compute_store(NLC - 1, seta)


_stage_c = pl.kernel(
    _stage_c_body,
    out_type=jax.ShapeDtypeStruct((N_LINKS,), _f32),
    mesh=_MESH,
    compiler_params=pltpu.CompilerParams(needs_layout_passes=False),
    scratch_types=(
        [pltpu.VMEM_SHARED((N_NODES,), _f32) for _ in range(3)]
        + 2 * ([pltpu.VMEM((CL,), _i32)]
               + [pltpu.VMEM((CL,), _f32) for _ in range(8)])
        + [pltpu.VMEM((CL,), _f32)]
        + [pltpu.VMEM((10_000,), _f32) for _ in range(5)]
        + [pltpu.SemaphoreType.DMA, pltpu.SemaphoreType.DMA]
    ),
)


def _stage_d_body(flux, lan, tr, area, dt16, out,
                  fsp, lan_v, g_v, tr_v, ar_v, dt_v, out_v, sem):
    s = lax.axis_index("s")
    w = _wid()

    _hbm_to_spmem(flux, fsp, g_v, s * (N_LINKS // NS), 10, csz=10_000)
    pltpu.sync_copy(dt16, dt_v)
    plsc.subcore_barrier()
    dtv = dt_v[...]
    iota16 = lax.iota(_i32, 16) * 16

    def chunk(i, carry):
        ch = w + i * NW

        @pl.when(ch < NCH)
        def _():
            noff = pl.multiple_of(ch * CN, 8)
            off = pl.multiple_of(ch * CN * LPN, 8)
            pltpu.sync_copy(lan.at[pl.ds(off, CN * LPN)], lan_v)
            pltpu.sync_copy(tr.at[pl.ds(noff, CN)], tr_v)
            pltpu.sync_copy(area.at[pl.ds(noff, CN)], ar_v)
            pltpu.async_copy(fsp.at[lan_v], g_v, sem).wait()

            def red(ii, carry2):
                b = ii * (16 * LPN)
                acc = jnp.zeros((16,), _f32)
                for k in range(LPN):
                    acc = acc + plsc.load_gather(g_v, [iota16 + (b + k)])
                vs = pl.ds(ii * 16, 16)
                a = ar_v[vs]
                nz = a != 0.0
                asafe = jnp.where(nz, a, 1.0)
                div = jnp.where(nz, acc / asafe, 0.0)
                out_v[vs] = tr_v[vs] + dtv * div
                return carry2

            lax.fori_loop(0, CN // 16, red, 0)
            pltpu.sync_copy(out_v, out.at[pl.ds(noff, CN)])
        return carry

    lax.fori_loop(0, (NCH + NW - 1) // NW, chunk, 0)


_stage_d = pl.kernel(
    _stage_d_body,
    out_type=jax.ShapeDtypeStruct((N_NODES,), _f32),
    mesh=_MESH,
    compiler_params=pltpu.CompilerParams(needs_layout_passes=False),
    scratch_types=(
        pltpu.VMEM_SHARED((N_LINKS,), _f32),
        pltpu.VMEM((CN * LPN,), _i32),
        pltpu.VMEM((CN * LPN,), _f32),
        pltpu.VMEM((CN,), _f32),
        pltpu.VMEM((CN,), _f32),
        pltpu.VMEM((16,), _f32),
        pltpu.VMEM((CN,), _f32),
        pltpu.SemaphoreType.DMA,
    ),
)


def kernel(velocity, tracer, node_x, node_y, cell_area_at_node, dt,
           node_at_link_head, node_at_link_tail, links_at_node,
           upwind_real_idx):
    hidx = node_at_link_head.astype(_i32)
    tidx = node_at_link_tail.astype(_i32)
    uidx = upwind_real_idx.astype(_i32)
    lan_flat = links_at_node.astype(_i32).reshape(-1)
    dt16 = jnp.broadcast_to(dt.astype(_f32), (16,))

    gx, gy, cen, dif, gox, goy = _stage_a(
        node_x, node_y, tracer, hidx, tidx, velocity)
    gxn, gyn = _stage_b(gx, gy, lan_flat)
    flux = _stage_c(node_x, node_y, tracer, gxn, gyn, uidx, velocity,
                    cen, dif, gox, goy)
    return _stage_d(flux, lan_flat, tracer, cell_area_at_node, dt16)


# speedup vs baseline: 211.8797x; 1.0162x over previous
import functools

import jax
import jax.numpy as jnp
from jax import lax
from jax.experimental import pallas as pl
from jax.experimental.pallas import tpu as pltpu
from jax.experimental.pallas import tpu_sc as plsc

N_NODES = 100_000
N_LINKS = 1_600_000
LPN = 16

NC = 2
NS = 16
NW = NC * NS

LW = N_LINKS // NW
CL = 2_000
NLC = LW // CL

CN = 800
NCH = N_NODES // CN

_MESH = plsc.VectorSubcoreMesh(
    core_axis_name="c", subcore_axis_name="s", num_cores=NC, num_subcores=NS)

_f32 = jnp.float32
_i32 = jnp.int32


def _wid():
    return lax.axis_index("s") * NC + lax.axis_index("c")


def _hbm_to_spmem(hbm_ref, sp_ref, vtmp, base, nchunks, csz=None):
    if csz is None:
        csz = vtmp.shape[0]

    def body(i, carry):
        o = pl.multiple_of(base + i * csz, 8)
        sl = pl.ds(o, csz)
        pltpu.sync_copy(hbm_ref.at[sl], vtmp.at[pl.ds(0, csz)])
        pltpu.sync_copy(vtmp.at[pl.ds(0, csz)], sp_ref.at[sl])
        return carry

    lax.fori_loop(0, nchunks, body, 0)


def _stage_a_body(nx, ny, tr, hidx, tidx, vel,
                  gx, gy, cen, dif, gox, goy,
                  xs, ys, ts,
                  hidx_a, tidx_a, vel_a, hx_a, hy_a, htr_a, tx_a, ty_a, ttr_a,
                  hidx_b, tidx_b, vel_b, hx_b, hy_b, htr_b, tx_b, ty_b, ttr_b,
                  gx_a, gy_a, cen_a, dif_a, gox_a, goy_a,
                  gx_b, gy_b, cen_b, dif_b, gox_b, goy_b,
                  stg_v,
                  sem_a, sem_b, sst_a, sst_b):
    s = lax.axis_index("s")
    base = _wid() * LW

    @pl.when(s < 4)
    def _stage_tables():
        base_t = s * 25_000
        _hbm_to_spmem(nx, xs, stg_v, base_t, 1)
        _hbm_to_spmem(ny, ys, stg_v, base_t, 1)
        _hbm_to_spmem(tr, ts, stg_v, base_t, 1)

    plsc.subcore_barrier()

    seta = (hidx_a, tidx_a, vel_a, hx_a, hy_a, htr_a, tx_a, ty_a, ttr_a,
            sem_a, gx_a, gy_a, cen_a, dif_a, gox_a, goy_a, sst_a)
    setb = (hidx_b, tidx_b, vel_b, hx_b, hy_b, htr_b, tx_b, ty_b, ttr_b,
            sem_b, gx_b, gy_b, cen_b, dif_b, gox_b, goy_b, sst_b)

    def load_fire(ci, st):
        hidx_v, tidx_v, vel_v, hx_v, hy_v, htr_v, tx_v, ty_v, ttr_v, sem = \
            st[:10]
        sl = pl.ds(pl.multiple_of(base + ci * CL, 8), CL)
        pltpu.sync_copy(hidx.at[sl], hidx_v)
        pltpu.sync_copy(tidx.at[sl], tidx_v)
        pltpu.sync_copy(vel.at[sl], vel_v)
        pltpu.async_copy(xs.at[hidx_v], hx_v, sem)
        pltpu.async_copy(ys.at[hidx_v], hy_v, sem)
        pltpu.async_copy(ts.at[hidx_v], htr_v, sem)
        pltpu.async_copy(xs.at[tidx_v], tx_v, sem)
        pltpu.async_copy(ys.at[tidx_v], ty_v, sem)
        pltpu.async_copy(ts.at[tidx_v], ttr_v, sem)

    def drain(st):
        sem = st[9]
        dummy = nx.at[pl.ds(0, CL)]
        for dst in st[3:9]:
            pltpu.make_async_copy(dummy, dst, sem).wait()

    def drain_stores(st):
        sst = st[16]
        dummy = nx.at[pl.ds(0, CL)]
        for dst in st[10:16]:
            pltpu.make_async_copy(dummy, dst, sst).wait()

    def compute_store(ci, st):
        _, _, vel_v, hx_v, hy_v, htr_v, tx_v, ty_v, ttr_v = st[:9]
        gx_v, gy_v, cen_v, dif_v, gox_v, goy_v, sst = st[10:]
        sl = pl.ds(pl.multiple_of(base + ci * CL, 8), CL)

        @pl.when(ci >= 2)
        def _():
            drain_stores(st)

        def vec(j, carry2):
            vs = pl.ds(j * 16, 16)
            hx = hx_v[vs]; hy = hy_v[vs]; htr = htr_v[vs]
            tx = tx_v[vs]; ty = ty_v[vs]; ttr = ttr_v[vs]
            v = vel_v[vs]
            dx = hx - tx
            dy = hy - ty
            l2 = jnp.maximum(dx * dx + dy * dy, 1e-18)
            dtr = (htr - ttr) / l2
            gx_v[vs] = dtr * dx
            gy_v[vs] = dtr * dy
            vpos = v >= 0.0
            cen_v[vs] = jnp.where(vpos, ttr, htr)
            dif_v[vs] = jnp.where(vpos, htr - ttr, ttr - htr)
            gox_v[vs] = jnp.where(vpos, 2.0 * hx - tx, 2.0 * tx - hx)
            goy_v[vs] = jnp.where(vpos, 2.0 * hy - ty, 2.0 * ty - hy)
            return carry2

        lax.fori_loop(0, CL // 16, vec, 0)
        pltpu.async_copy(gx_v, gx.at[sl], sst)
        pltpu.async_copy(gy_v, gy.at[sl], sst)
        pltpu.async_copy(cen_v, cen.at[sl], sst)
        pltpu.async_copy(dif_v, dif.at[sl], sst)
        pltpu.async_copy(gox_v, gox.at[sl], sst)
        pltpu.async_copy(goy_v, goy.at[sl], sst)

    load_fire(0, seta)

    def pipe(i, carry):
        c0 = i * 2
        load_fire(c0 + 1, setb)
        drain(seta)
        compute_store(c0, seta)
        load_fire(c0 + 2, seta)
        drain(setb)
        compute_store(c0 + 1, setb)
        return carry

    lax.fori_loop(0, NLC // 2, pipe, 0)
    drain(seta)
    compute_store(NLC - 1, seta)
    drain_stores(seta)
    drain_stores(setb)


_stage_a = pl.kernel(
    _stage_a_body,
    out_type=tuple(jax.ShapeDtypeStruct((N_LINKS,), _f32) for _ in range(6)),
    mesh=_MESH,
    compiler_params=pltpu.CompilerParams(needs_layout_passes=False),
    scratch_types=(
        [pltpu.VMEM_SHARED((N_NODES,), _f32) for _ in range(3)]
        + 2 * ([pltpu.VMEM((CL,), _i32) for _ in range(2)]
               + [pltpu.VMEM((CL,), _f32) for _ in range(7)])
        + [pltpu.VMEM((CL,), _f32) for _ in range(12)]
        + [pltpu.VMEM((25_000,), _f32)]
        + [pltpu.SemaphoreType.DMA for _ in range(4)]
    ),
)


def _stage_b_body(gx, gy, lan, gxn, gyn,
                  fsp, lan_v, g_v, out_v, sem):
    c = lax.axis_index("c")
    s = lax.axis_index("s")

    def run(field, out):
        _hbm_to_spmem(field, fsp, g_v, s * (N_LINKS // NS), 10, csz=10_000)
        plsc.subcore_barrier()
        iota16 = lax.iota(_i32, 16) * 16

        def chunk(i, carry):
            ch = s + i * NS

            @pl.when(ch < NCH)
            def _():
                off = pl.multiple_of(ch * CN * LPN, 8)
                pltpu.sync_copy(lan.at[pl.ds(off, CN * LPN)], lan_v)
                pltpu.async_copy(fsp.at[lan_v], g_v, sem).wait()

                def red(ii, carry2):
                    b = ii * (16 * LPN)
                    acc = jnp.zeros((16,), _f32)
                    for k in range(LPN):
                        acc = acc + plsc.load_gather(g_v, [iota16 + (b + k)])
                    out_v[pl.ds(ii * 16, 16)] = acc * (1.0 / LPN)
                    return carry2

                lax.fori_loop(0, CN // 16, red, 0)
                pltpu.sync_copy(out_v, out.at[pl.ds(pl.multiple_of(ch * CN, 8), CN)])
            return carry

        lax.fori_loop(0, (NCH + NS - 1) // NS, chunk, 0)

    @pl.when(c == 0)
    def _():
        run(gx, gxn)

    @pl.when(c == 1)
    def _():
        run(gy, gyn)


_stage_b = pl.kernel(
    _stage_b_body,
    out_type=tuple(jax.ShapeDtypeStruct((N_NODES,), _f32) for _ in range(2)),
    mesh=_MESH,
    compiler_params=pltpu.CompilerParams(needs_layout_passes=False),
    scratch_types=(
        pltpu.VMEM_SHARED((N_LINKS,), _f32),
        pltpu.VMEM((CN * LPN,), _i32),
        pltpu.VMEM((CN * LPN,), _f32),
        pltpu.VMEM((CN,), _f32),
        pltpu.SemaphoreType.DMA,
    ),
)


def _stage_c_body(nx, ny, tr, gxn, gyn, uidx, vel, cen, dif, gox, goy,
                  flux,
                  ps, gxs, gys,
                  uidx_a, vel_a, cen_a, dif_a, gox_a, goy_a,
                  up_a, ugx_a, ugy_a,
                  uidx_b, vel_b, cen_b, dif_b, gox_b, goy_b,
                  up_b, ugx_b, ugy_b,
                  flux_a, flux_b,
                  sx_v, sy_v, st_v, sgx_v, sgy_v,
                  sem_a, sem_b, sst_a, sst_b):
    s = lax.axis_index("s")

    @pl.when(s < 10)
    def _stage_tables():
        o = pl.multiple_of(s * 10_000, 8)
        slt = pl.ds(o, 10_000)
        pltpu.sync_copy(nx.at[slt], sx_v)
        pltpu.sync_copy(ny.at[slt], sy_v)
        pltpu.sync_copy(tr.at[slt], st_v)
        pltpu.sync_copy(gxn.at[slt], sgx_v)
        pltpu.sync_copy(gyn.at[slt], sgy_v)

        def pbody(j, carry):
            vs = pl.ds(j * 16, 16)
            st_v[vs] = (st_v[vs] + sx_v[vs] * sgx_v[vs]
                        + sy_v[vs] * sgy_v[vs])
            return carry

        lax.fori_loop(0, 10_000 // 16, pbody, 0)
        pltpu.sync_copy(st_v, ps.at[slt])
        pltpu.sync_copy(sgx_v, gxs.at[slt])
        pltpu.sync_copy(sgy_v, gys.at[slt])

    plsc.subcore_barrier()
    base = _wid() * LW

    seta = (uidx_a, vel_a, cen_a, dif_a, gox_a, goy_a, up_a, ugx_a, ugy_a,
            sem_a, flux_a, sst_a)
    setb = (uidx_b, vel_b, cen_b, dif_b, gox_b, goy_b, up_b, ugx_b, ugy_b,
            sem_b, flux_b, sst_b)

    def load_fire(ci, st):
        (uidx_v, vel_v, cen_v, dif_v, gox_v, goy_v, up_v, ugx_v, ugy_v,
         sem) = st[:10]
        sl = pl.ds(pl.multiple_of(base + ci * CL, 8), CL)
        pltpu.sync_copy(uidx.at[sl], uidx_v)
        pltpu.sync_copy(vel.at[sl], vel_v)
        pltpu.sync_copy(cen.at[sl], cen_v)
        pltpu.sync_copy(dif.at[sl], dif_v)
        pltpu.sync_copy(gox.at[sl], gox_v)
        pltpu.sync_copy(goy.at[sl], goy_v)
        pltpu.async_copy(ps.at[uidx_v], up_v, sem)
        pltpu.async_copy(gxs.at[uidx_v], ugx_v, sem)
        pltpu.async_copy(gys.at[uidx_v], ugy_v, sem)

    def drain(st):
        sem = st[9]
        dummy = nx.at[pl.ds(0, CL)]
        for dst in st[6:9]:
            pltpu.make_async_copy(dummy, dst, sem).wait()

    def compute_store(ci, st):
        _, vel_v, cen_v, dif_v, gox_v, goy_v, up_v, ugx_v, ugy_v = st[:9]
        flux_v, sst = st[10:]
        sl = pl.ds(pl.multiple_of(base + ci * CL, 8), CL)

        @pl.when(ci >= 2)
        def _():
            pltpu.make_async_copy(nx.at[pl.ds(0, CL)], flux_v, sst).wait()

        def vec(j, carry2):
            vs = pl.ds(j * 16, 16)
            up = (up_v[vs] - gox_v[vs] * ugx_v[vs]
                  - goy_v[vs] * ugy_v[vs])
            ce = cen_v[vs]
            di = dif_v[vs]
            nz = di != 0.0
            den = jnp.where(nz, di, 1.0)
            r = jnp.where(nz, (ce - up) / den, 0.0)
            ar = jnp.abs(r)
            phi = (r + ar) / (1.0 + ar)
            flux_v[vs] = vel_v[vs] * (ce + 0.5 * phi * di)
            return carry2

        lax.fori_loop(0, CL // 16, vec, 0)
        pltpu.async_copy(flux_v, flux.at[sl], sst)

    load_fire(0, seta)

    def pipe(i, carry):
        c0 = i * 2
        load_fire(c0 + 1, setb)
        drain(seta)
        compute_store(c0, seta)
        load_fire(c0 + 2, seta)
        drain(setb)
        compute_store(c0 + 1, setb)
        return carry

    lax.fori_loop(0, NLC // 2, pipe, 0)
    drain(seta)
    compute_store(NLC - 1, seta)
    pltpu.make_async_copy(nx.at[pl.ds(0, CL)], flux_a, sst_a).wait()
    pltpu.make_async_copy(nx.at[pl.ds(0, CL)], flux_b, sst_b).wait()


_stage_c = pl.kernel(
    _stage_c_body,
    out_type=jax.ShapeDtypeStruct((N_LINKS,), _f32),
    mesh=_MESH,
    compiler_params=pltpu.CompilerParams(needs_layout_passes=False),
    scratch_types=(
        [pltpu.VMEM_SHARED((N_NODES,), _f32) for _ in range(3)]
        + 2 * ([pltpu.VMEM((CL,), _i32)]
               + [pltpu.VMEM((CL,), _f32) for _ in range(8)])
        + [pltpu.VMEM((CL,), _f32) for _ in range(2)]
        + [pltpu.VMEM((10_000,), _f32) for _ in range(5)]
        + [pltpu.SemaphoreType.DMA for _ in range(4)]
    ),
)


def _stage_d_body(flux, lan, tr, area, dt16, out,
                  fsp, lan_v, g_v, tr_v, ar_v, dt_v, out_v, sem):
    s = lax.axis_index("s")
    w = _wid()

    _hbm_to_spmem(flux, fsp, g_v, s * (N_LINKS // NS), 10, csz=10_000)
    pltpu.sync_copy(dt16, dt_v)
    plsc.subcore_barrier()
    dtv = dt_v[...]
    iota16 = lax.iota(_i32, 16) * 16

    def chunk(i, carry):
        ch = w + i * NW

        @pl.when(ch < NCH)
        def _():
            noff = pl.multiple_of(ch * CN, 8)
            off = pl.multiple_of(ch * CN * LPN, 8)
            pltpu.sync_copy(lan.at[pl.ds(off, CN * LPN)], lan_v)
            pltpu.sync_copy(tr.at[pl.ds(noff, CN)], tr_v)
            pltpu.sync_copy(area.at[pl.ds(noff, CN)], ar_v)
            pltpu.async_copy(fsp.at[lan_v], g_v, sem).wait()

            def red(ii, carry2):
                b = ii * (16 * LPN)
                acc = jnp.zeros((16,), _f32)
                for k in range(LPN):
                    acc = acc + plsc.load_gather(g_v, [iota16 + (b + k)])
                vs = pl.ds(ii * 16, 16)
                a = ar_v[vs]
                nz = a != 0.0
                asafe = jnp.where(nz, a, 1.0)
                div = jnp.where(nz, acc / asafe, 0.0)
                out_v[vs] = tr_v[vs] + dtv * div
                return carry2

            lax.fori_loop(0, CN // 16, red, 0)
            pltpu.sync_copy(out_v, out.at[pl.ds(noff, CN)])
        return carry

    lax.fori_loop(0, (NCH + NW - 1) // NW, chunk, 0)


_stage_d = pl.kernel(
    _stage_d_body,
    out_type=jax.ShapeDtypeStruct((N_NODES,), _f32),
    mesh=_MESH,
    compiler_params=pltpu.CompilerParams(needs_layout_passes=False),
    scratch_types=(
        pltpu.VMEM_SHARED((N_LINKS,), _f32),
        pltpu.VMEM((CN * LPN,), _i32),
        pltpu.VMEM((CN * LPN,), _f32),
        pltpu.VMEM((CN,), _f32),
        pltpu.VMEM((CN,), _f32),
        pltpu.VMEM((16,), _f32),
        pltpu.VMEM((CN,), _f32),
        pltpu.SemaphoreType.DMA,
    ),
)


def kernel(velocity, tracer, node_x, node_y, cell_area_at_node, dt,
           node_at_link_head, node_at_link_tail, links_at_node,
           upwind_real_idx):
    hidx = node_at_link_head.astype(_i32)
    tidx = node_at_link_tail.astype(_i32)
    uidx = upwind_real_idx.astype(_i32)
    lan_flat = links_at_node.astype(_i32).reshape(-1)
    dt16 = jnp.broadcast_to(dt.astype(_f32), (16,))

    gx, gy, cen, dif, gox, goy = _stage_a(
        node_x, node_y, tracer, hidx, tidx, velocity)
    gxn, gyn = _stage_b(gx, gy, lan_flat)
    flux = _stage_c(node_x, node_y, tracer, gxn, gyn, uidx, velocity,
                    cen, dif, gox, goy)
    return _stage_d(flux, lan_flat, tracer, cell_area_at_node, dt16)


# speedup vs baseline: 256.7427x; 1.2117x over previous
import functools

import jax
import jax.numpy as jnp
from jax import lax
from jax.experimental import pallas as pl
from jax.experimental.pallas import tpu as pltpu
from jax.experimental.pallas import tpu_sc as plsc

N_NODES = 100_000
N_LINKS = 1_600_000
LPN = 16

NC = 2
NS = 16
NW = NC * NS

LW = N_LINKS // NW
CL = 2_000
NLC = LW // CL

CN = 800
NCH = N_NODES // CN

_MESH = plsc.VectorSubcoreMesh(
    core_axis_name="c", subcore_axis_name="s", num_cores=NC, num_subcores=NS)

_f32 = jnp.float32
_i32 = jnp.int32


def _wid():
    return lax.axis_index("s") * NC + lax.axis_index("c")


def _hbm_to_spmem(hbm_ref, sp_ref, vtmp, base, nchunks, csz=None):
    if csz is None:
        csz = vtmp.shape[0]

    def body(i, carry):
        o = pl.multiple_of(base + i * csz, 8)
        sl = pl.ds(o, csz)
        pltpu.sync_copy(hbm_ref.at[sl], vtmp.at[pl.ds(0, csz)])
        pltpu.sync_copy(vtmp.at[pl.ds(0, csz)], sp_ref.at[sl])
        return carry

    lax.fori_loop(0, nchunks, body, 0)


def _stage_a_body(nx, ny, tr, hidx, tidx, vel,
                  gx, gy, cen, dif, gox, goy,
                  xs, ys, ts,
                  hidx_a, tidx_a, vel_a, hx_a, hy_a, htr_a, tx_a, ty_a, ttr_a,
                  hidx_b, tidx_b, vel_b, hx_b, hy_b, htr_b, tx_b, ty_b, ttr_b,
                  gx_a, gy_a, cen_a, dif_a, gox_a, goy_a,
                  gx_b, gy_b, cen_b, dif_b, gox_b, goy_b,
                  stg_v,
                  sem_a, sem_b, sst_a, sst_b):
    s = lax.axis_index("s")
    base = _wid() * LW

    @pl.when(s < 4)
    def _stage_tables():
        base_t = s * 25_000
        _hbm_to_spmem(nx, xs, stg_v, base_t, 1)
        _hbm_to_spmem(ny, ys, stg_v, base_t, 1)
        _hbm_to_spmem(tr, ts, stg_v, base_t, 1)

    plsc.subcore_barrier()

    seta = (hidx_a, tidx_a, vel_a, hx_a, hy_a, htr_a, tx_a, ty_a, ttr_a,
            sem_a, gx_a, gy_a, cen_a, dif_a, gox_a, goy_a, sst_a)
    setb = (hidx_b, tidx_b, vel_b, hx_b, hy_b, htr_b, tx_b, ty_b, ttr_b,
            sem_b, gx_b, gy_b, cen_b, dif_b, gox_b, goy_b, sst_b)

    def load_fire(ci, st):
        hidx_v, tidx_v, vel_v, hx_v, hy_v, htr_v, tx_v, ty_v, ttr_v, sem = \
            st[:10]
        sl = pl.ds(pl.multiple_of(base + ci * CL, 8), CL)
        pltpu.sync_copy(hidx.at[sl], hidx_v)
        pltpu.sync_copy(tidx.at[sl], tidx_v)
        pltpu.sync_copy(vel.at[sl], vel_v)
        pltpu.async_copy(xs.at[hidx_v], hx_v, sem)
        pltpu.async_copy(ys.at[hidx_v], hy_v, sem)
        pltpu.async_copy(ts.at[hidx_v], htr_v, sem)
        pltpu.async_copy(xs.at[tidx_v], tx_v, sem)
        pltpu.async_copy(ys.at[tidx_v], ty_v, sem)
        pltpu.async_copy(ts.at[tidx_v], ttr_v, sem)

    def drain(st):
        sem = st[9]
        dummy = nx.at[pl.ds(0, CL)]
        for dst in st[3:9]:
            pltpu.make_async_copy(dummy, dst, sem).wait()

    def drain_stores(st):
        sst = st[16]
        dummy = nx.at[pl.ds(0, CL)]
        for dst in st[10:16]:
            pltpu.make_async_copy(dummy, dst, sst).wait()

    def compute_store(ci, st):
        _, _, vel_v, hx_v, hy_v, htr_v, tx_v, ty_v, ttr_v = st[:9]
        gx_v, gy_v, cen_v, dif_v, gox_v, goy_v, sst = st[10:]
        sl = pl.ds(pl.multiple_of(base + ci * CL, 8), CL)

        @pl.when(ci >= 2)
        def _():
            drain_stores(st)

        def vec(j, carry2):
            vs = pl.ds(j * 16, 16)
            hx = hx_v[vs]; hy = hy_v[vs]; htr = htr_v[vs]
            tx = tx_v[vs]; ty = ty_v[vs]; ttr = ttr_v[vs]
            v = vel_v[vs]
            dx = hx - tx
            dy = hy - ty
            l2 = jnp.maximum(dx * dx + dy * dy, 1e-18)
            dtr = (htr - ttr) / l2
            gx_v[vs] = dtr * dx
            gy_v[vs] = dtr * dy
            vpos = v >= 0.0
            cen_v[vs] = jnp.where(vpos, ttr, htr)
            dif_v[vs] = jnp.where(vpos, htr - ttr, ttr - htr)
            gox_v[vs] = jnp.where(vpos, 2.0 * hx - tx, 2.0 * tx - hx)
            goy_v[vs] = jnp.where(vpos, 2.0 * hy - ty, 2.0 * ty - hy)
            return carry2

        lax.fori_loop(0, CL // 16, vec, 0)
        pltpu.async_copy(gx_v, gx.at[sl], sst)
        pltpu.async_copy(gy_v, gy.at[sl], sst)
        pltpu.async_copy(cen_v, cen.at[sl], sst)
        pltpu.async_copy(dif_v, dif.at[sl], sst)
        pltpu.async_copy(gox_v, gox.at[sl], sst)
        pltpu.async_copy(goy_v, goy.at[sl], sst)

    load_fire(0, seta)

    def pipe(i, carry):
        c0 = i * 2
        load_fire(c0 + 1, setb)
        drain(seta)
        compute_store(c0, seta)
        load_fire(c0 + 2, seta)
        drain(setb)
        compute_store(c0 + 1, setb)
        return carry

    lax.fori_loop(0, NLC // 2, pipe, 0)
    drain(seta)
    compute_store(NLC - 1, seta)
    drain_stores(seta)
    drain_stores(setb)


_stage_a = pl.kernel(
    _stage_a_body,
    out_type=tuple(jax.ShapeDtypeStruct((N_LINKS,), _f32) for _ in range(6)),
    mesh=_MESH,
    compiler_params=pltpu.CompilerParams(needs_layout_passes=False),
    scratch_types=(
        [pltpu.VMEM_SHARED((N_NODES,), _f32) for _ in range(3)]
        + 2 * ([pltpu.VMEM((CL,), _i32) for _ in range(2)]
               + [pltpu.VMEM((CL,), _f32) for _ in range(7)])
        + [pltpu.VMEM((CL,), _f32) for _ in range(12)]
        + [pltpu.VMEM((25_000,), _f32)]
        + [pltpu.SemaphoreType.DMA for _ in range(4)]
    ),
)


def _stage_b_body(gx, gy, lan, gxn, gyn,
                  fsp, lan_v, g_v, out_v, sem):
    c = lax.axis_index("c")
    s = lax.axis_index("s")

    def run(field, out):
        _hbm_to_spmem(field, fsp, g_v, s * (N_LINKS // NS), 10, csz=10_000)
        plsc.subcore_barrier()
        iota16 = lax.iota(_i32, 16) * 16

        def chunk(i, carry):
            ch = s + i * NS

            @pl.when(ch < NCH)
            def _():
                off = pl.multiple_of(ch * CN * LPN, 8)
                pltpu.sync_copy(lan.at[pl.ds(off, CN * LPN)], lan_v)
                pltpu.async_copy(fsp.at[lan_v], g_v, sem).wait()

                def red(ii, carry2):
                    b = ii * (16 * LPN)
                    acc = jnp.zeros((16,), _f32)
                    for k in range(LPN):
                        acc = acc + plsc.load_gather(g_v, [iota16 + (b + k)])
                    out_v[pl.ds(ii * 16, 16)] = acc * (1.0 / LPN)
                    return carry2

                lax.fori_loop(0, CN // 16, red, 0)
                pltpu.sync_copy(out_v, out.at[pl.ds(pl.multiple_of(ch * CN, 8), CN)])
            return carry

        lax.fori_loop(0, (NCH + NS - 1) // NS, chunk, 0)

    @pl.when(c == 0)
    def _():
        run(gx, gxn)

    @pl.when(c == 1)
    def _():
        run(gy, gyn)


_stage_b = pl.kernel(
    _stage_b_body,
    out_type=tuple(jax.ShapeDtypeStruct((N_NODES,), _f32) for _ in range(2)),
    mesh=_MESH,
    compiler_params=pltpu.CompilerParams(needs_layout_passes=False),
    scratch_types=(
        pltpu.VMEM_SHARED((N_LINKS,), _f32),
        pltpu.VMEM((CN * LPN,), _i32),
        pltpu.VMEM((CN * LPN,), _f32),
        pltpu.VMEM((CN,), _f32),
        pltpu.SemaphoreType.DMA,
    ),
)


def _stage_c_body(nx, ny, tr, gxn, gyn, uidx, vel, cen, dif, gox, goy,
                  flux,
                  ps, gxs, gys,
                  uidx_a, vel_a, cen_a, dif_a, gox_a, goy_a,
                  up_a, ugx_a, ugy_a,
                  uidx_b, vel_b, cen_b, dif_b, gox_b, goy_b,
                  up_b, ugx_b, ugy_b,
                  flux_a, flux_b,
                  sx_v, sy_v, st_v, sgx_v, sgy_v,
                  sem_a, sem_b, sst_a, sst_b, slin_a, slin_b):
    s = lax.axis_index("s")

    @pl.when(s < 10)
    def _stage_tables():
        o = pl.multiple_of(s * 10_000, 8)
        slt = pl.ds(o, 10_000)
        pltpu.sync_copy(nx.at[slt], sx_v)
        pltpu.sync_copy(ny.at[slt], sy_v)
        pltpu.sync_copy(tr.at[slt], st_v)
        pltpu.sync_copy(gxn.at[slt], sgx_v)
        pltpu.sync_copy(gyn.at[slt], sgy_v)

        def pbody(j, carry):
            vs = pl.ds(j * 16, 16)
            st_v[vs] = (st_v[vs] + sx_v[vs] * sgx_v[vs]
                        + sy_v[vs] * sgy_v[vs])
            return carry

        lax.fori_loop(0, 10_000 // 16, pbody, 0)
        pltpu.sync_copy(st_v, ps.at[slt])
        pltpu.sync_copy(sgx_v, gxs.at[slt])
        pltpu.sync_copy(sgy_v, gys.at[slt])

    plsc.subcore_barrier()
    base = _wid() * LW

    seta = (uidx_a, vel_a, cen_a, dif_a, gox_a, goy_a, up_a, ugx_a, ugy_a,
            sem_a, flux_a, sst_a, slin_a)
    setb = (uidx_b, vel_b, cen_b, dif_b, gox_b, goy_b, up_b, ugx_b, ugy_b,
            sem_b, flux_b, sst_b, slin_b)

    def load_fire(ci, st):
        (uidx_v, vel_v, cen_v, dif_v, gox_v, goy_v, up_v, ugx_v, ugy_v,
         sem) = st[:10]
        slin = st[12]
        sl = pl.ds(pl.multiple_of(base + ci * CL, 8), CL)
        pltpu.sync_copy(uidx.at[sl], uidx_v)
        pltpu.async_copy(vel.at[sl], vel_v, slin)
        pltpu.async_copy(cen.at[sl], cen_v, slin)
        pltpu.async_copy(dif.at[sl], dif_v, slin)
        pltpu.async_copy(gox.at[sl], gox_v, slin)
        pltpu.async_copy(goy.at[sl], goy_v, slin)
        pltpu.async_copy(ps.at[uidx_v], up_v, sem)
        pltpu.async_copy(gxs.at[uidx_v], ugx_v, sem)
        pltpu.async_copy(gys.at[uidx_v], ugy_v, sem)

    def drain(st):
        sem = st[9]
        slin = st[12]
        dummy = nx.at[pl.ds(0, CL)]
        for dst in st[1:6]:
            pltpu.make_async_copy(dummy, dst, slin).wait()
        for dst in st[6:9]:
            pltpu.make_async_copy(dummy, dst, sem).wait()

    def compute_store(ci, st):
        _, vel_v, cen_v, dif_v, gox_v, goy_v, up_v, ugx_v, ugy_v = st[:9]
        flux_v, sst = st[10:12]
        sl = pl.ds(pl.multiple_of(base + ci * CL, 8), CL)

        @pl.when(ci >= 2)
        def _():
            pltpu.make_async_copy(nx.at[pl.ds(0, CL)], flux_v, sst).wait()

        def vec(j, carry2):
            vs = pl.ds(j * 16, 16)
            up = (up_v[vs] - gox_v[vs] * ugx_v[vs]
                  - goy_v[vs] * ugy_v[vs])
            ce = cen_v[vs]
            di = dif_v[vs]
            nz = di != 0.0
            den = jnp.where(nz, di, 1.0)
            r = jnp.where(nz, (ce - up) / den, 0.0)
            ar = jnp.abs(r)
            phi = (r + ar) / (1.0 + ar)
            flux_v[vs] = vel_v[vs] * (ce + 0.5 * phi * di)
            return carry2

        lax.fori_loop(0, CL // 16, vec, 0)
        pltpu.async_copy(flux_v, flux.at[sl], sst)

    load_fire(0, seta)

    def pipe(i, carry):
        c0 = i * 2
        load_fire(c0 + 1, setb)
        drain(seta)
        compute_store(c0, seta)
        load_fire(c0 + 2, seta)
        drain(setb)
        compute_store(c0 + 1, setb)
        return carry

    lax.fori_loop(0, NLC // 2, pipe, 0)
    drain(seta)
    compute_store(NLC - 1, seta)
    pltpu.make_async_copy(nx.at[pl.ds(0, CL)], flux_a, sst_a).wait()
    pltpu.make_async_copy(nx.at[pl.ds(0, CL)], flux_b, sst_b).wait()


_stage_c = pl.kernel(
    _stage_c_body,
    out_type=jax.ShapeDtypeStruct((N_LINKS,), _f32),
    mesh=_MESH,
    compiler_params=pltpu.CompilerParams(needs_layout_passes=False),
    scratch_types=(
        [pltpu.VMEM_SHARED((N_NODES,), _f32) for _ in range(3)]
        + 2 * ([pltpu.VMEM((CL,), _i32)]
               + [pltpu.VMEM((CL,), _f32) for _ in range(8)])
        + [pltpu.VMEM((CL,), _f32) for _ in range(2)]
        + [pltpu.VMEM((10_000,), _f32) for _ in range(5)]
        + [pltpu.SemaphoreType.DMA for _ in range(6)]
    ),
)


def _stage_d_body(flux, lan, tr, area, dt16, out,
                  fsp, lan_v, g_v, tr_v, ar_v, dt_v, out_v, sem):
    s = lax.axis_index("s")
    w = _wid()

    _hbm_to_spmem(flux, fsp, g_v, s * (N_LINKS // NS), 10, csz=10_000)
    pltpu.sync_copy(dt16, dt_v)
    plsc.subcore_barrier()
    dtv = dt_v[...]
    iota16 = lax.iota(_i32, 16) * 16

    def chunk(i, carry):
        ch = w + i * NW

        @pl.when(ch < NCH)
        def _():
            noff = pl.multiple_of(ch * CN, 8)
            off = pl.multiple_of(ch * CN * LPN, 8)
            pltpu.sync_copy(lan.at[pl.ds(off, CN * LPN)], lan_v)
            pltpu.sync_copy(tr.at[pl.ds(noff, CN)], tr_v)
            pltpu.sync_copy(area.at[pl.ds(noff, CN)], ar_v)
            pltpu.async_copy(fsp.at[lan_v], g_v, sem).wait()

            def red(ii, carry2):
                b = ii * (16 * LPN)
                acc = jnp.zeros((16,), _f32)
                for k in range(LPN):
                    acc = acc + plsc.load_gather(g_v, [iota16 + (b + k)])
                vs = pl.ds(ii * 16, 16)
                a = ar_v[vs]
                nz = a != 0.0
                asafe = jnp.where(nz, a, 1.0)
                div = jnp.where(nz, acc / asafe, 0.0)
                out_v[vs] = tr_v[vs] + dtv * div
                return carry2

            lax.fori_loop(0, CN // 16, red, 0)
            pltpu.sync_copy(out_v, out.at[pl.ds(noff, CN)])
        return carry

    lax.fori_loop(0, (NCH + NW - 1) // NW, chunk, 0)


_stage_d = pl.kernel(
    _stage_d_body,
    out_type=jax.ShapeDtypeStruct((N_NODES,), _f32),
    mesh=_MESH,
    compiler_params=pltpu.CompilerParams(needs_layout_passes=False),
    scratch_types=(
        pltpu.VMEM_SHARED((N_LINKS,), _f32),
        pltpu.VMEM((CN * LPN,), _i32),
        pltpu.VMEM((CN * LPN,), _f32),
        pltpu.VMEM((CN,), _f32),
        pltpu.VMEM((CN,), _f32),
        pltpu.VMEM((16,), _f32),
        pltpu.VMEM((CN,), _f32),
        pltpu.SemaphoreType.DMA,
    ),
)


def kernel(velocity, tracer, node_x, node_y, cell_area_at_node, dt,
           node_at_link_head, node_at_link_tail, links_at_node,
           upwind_real_idx):
    hidx = node_at_link_head.astype(_i32)
    tidx = node_at_link_tail.astype(_i32)
    uidx = upwind_real_idx.astype(_i32)
    lan_flat = links_at_node.astype(_i32).reshape(-1)
    dt16 = jnp.broadcast_to(dt.astype(_f32), (16,))

    gx, gy, cen, dif, gox, goy = _stage_a(
        node_x, node_y, tracer, hidx, tidx, velocity)
    gxn, gyn = _stage_b(gx, gy, lan_flat)
    flux = _stage_c(node_x, node_y, tracer, gxn, gyn, uidx, velocity,
                    cen, dif, gox, goy)
    return _stage_d(flux, lan_flat, tracer, cell_area_at_node, dt16)


# speedup vs baseline: 281.0919x; 1.0948x over previous
import functools

import jax
import jax.numpy as jnp
from jax import lax
from jax.experimental import pallas as pl
from jax.experimental.pallas import tpu as pltpu
from jax.experimental.pallas import tpu_sc as plsc

N_NODES = 100_000
N_LINKS = 1_600_000
LPN = 16

NC = 2
NS = 16
NW = NC * NS

LW = N_LINKS // NW
CL = 2_000
NLC = LW // CL

CN = 800
NCH = N_NODES // CN

_MESH = plsc.VectorSubcoreMesh(
    core_axis_name="c", subcore_axis_name="s", num_cores=NC, num_subcores=NS)

_f32 = jnp.float32
_i32 = jnp.int32


def _wid():
    return lax.axis_index("s") * NC + lax.axis_index("c")


def _hbm_to_spmem(hbm_ref, sp_ref, vtmp, base, nchunks, csz=None):
    if csz is None:
        csz = vtmp.shape[0]

    def body(i, carry):
        o = pl.multiple_of(base + i * csz, 8)
        sl = pl.ds(o, csz)
        pltpu.sync_copy(hbm_ref.at[sl], vtmp.at[pl.ds(0, csz)])
        pltpu.sync_copy(vtmp.at[pl.ds(0, csz)], sp_ref.at[sl])
        return carry

    lax.fori_loop(0, nchunks, body, 0)


def _stage_a_body(nx, ny, tr, hidx, tidx, vel,
                  gx, gy, cen, dif, gox, goy,
                  xs, ys, ts,
                  hidx_a, tidx_a, vel_a, hx_a, hy_a, htr_a, tx_a, ty_a, ttr_a,
                  hidx_b, tidx_b, vel_b, hx_b, hy_b, htr_b, tx_b, ty_b, ttr_b,
                  gx_a, gy_a, cen_a, dif_a, gox_a, goy_a,
                  gx_b, gy_b, cen_b, dif_b, gox_b, goy_b,
                  stg_v,
                  sem_a, sem_b, sst_a, sst_b):
    s = lax.axis_index("s")
    base = _wid() * LW

    @pl.when(s < 4)
    def _stage_tables():
        base_t = s * 25_000
        _hbm_to_spmem(nx, xs, stg_v, base_t, 1)
        _hbm_to_spmem(ny, ys, stg_v, base_t, 1)
        _hbm_to_spmem(tr, ts, stg_v, base_t, 1)

    plsc.subcore_barrier()

    seta = (hidx_a, tidx_a, vel_a, hx_a, hy_a, htr_a, tx_a, ty_a, ttr_a,
            sem_a, gx_a, gy_a, cen_a, dif_a, gox_a, goy_a, sst_a)
    setb = (hidx_b, tidx_b, vel_b, hx_b, hy_b, htr_b, tx_b, ty_b, ttr_b,
            sem_b, gx_b, gy_b, cen_b, dif_b, gox_b, goy_b, sst_b)

    def load_fire(ci, st):
        hidx_v, tidx_v, vel_v, hx_v, hy_v, htr_v, tx_v, ty_v, ttr_v, sem = \
            st[:10]
        sl = pl.ds(pl.multiple_of(base + ci * CL, 8), CL)
        pltpu.sync_copy(hidx.at[sl], hidx_v)
        pltpu.sync_copy(tidx.at[sl], tidx_v)
        pltpu.sync_copy(vel.at[sl], vel_v)
        pltpu.async_copy(xs.at[hidx_v], hx_v, sem)
        pltpu.async_copy(ys.at[hidx_v], hy_v, sem)
        pltpu.async_copy(ts.at[hidx_v], htr_v, sem)
        pltpu.async_copy(xs.at[tidx_v], tx_v, sem)
        pltpu.async_copy(ys.at[tidx_v], ty_v, sem)
        pltpu.async_copy(ts.at[tidx_v], ttr_v, sem)

    def drain(st):
        sem = st[9]
        dummy = nx.at[pl.ds(0, CL)]
        for dst in st[3:9]:
            pltpu.make_async_copy(dummy, dst, sem).wait()

    def drain_stores(st):
        sst = st[16]
        dummy = nx.at[pl.ds(0, CL)]
        for dst in st[10:16]:
            pltpu.make_async_copy(dummy, dst, sst).wait()

    def compute_store(ci, st):
        _, _, vel_v, hx_v, hy_v, htr_v, tx_v, ty_v, ttr_v = st[:9]
        gx_v, gy_v, cen_v, dif_v, gox_v, goy_v, sst = st[10:]
        sl = pl.ds(pl.multiple_of(base + ci * CL, 8), CL)

        @pl.when(ci >= 2)
        def _():
            drain_stores(st)

        def vec(j, carry2):
            vs = pl.ds(j * 16, 16)
            hx = hx_v[vs]; hy = hy_v[vs]; htr = htr_v[vs]
            tx = tx_v[vs]; ty = ty_v[vs]; ttr = ttr_v[vs]
            v = vel_v[vs]
            dx = hx - tx
            dy = hy - ty
            l2 = jnp.maximum(dx * dx + dy * dy, 1e-18)
            dtr = (htr - ttr) / l2
            gx_v[vs] = dtr * dx
            gy_v[vs] = dtr * dy
            vpos = v >= 0.0
            cen_v[vs] = jnp.where(vpos, ttr, htr)
            dif_v[vs] = jnp.where(vpos, htr - ttr, ttr - htr)
            gox_v[vs] = jnp.where(vpos, 2.0 * hx - tx, 2.0 * tx - hx)
            goy_v[vs] = jnp.where(vpos, 2.0 * hy - ty, 2.0 * ty - hy)
            return carry2

        lax.fori_loop(0, CL // 16, vec, 0)
        pltpu.async_copy(gx_v, gx.at[sl], sst)
        pltpu.async_copy(gy_v, gy.at[sl], sst)
        pltpu.async_copy(cen_v, cen.at[sl], sst)
        pltpu.async_copy(dif_v, dif.at[sl], sst)
        pltpu.async_copy(gox_v, gox.at[sl], sst)
        pltpu.async_copy(goy_v, goy.at[sl], sst)

    load_fire(0, seta)

    def pipe(i, carry):
        c0 = i * 2
        load_fire(c0 + 1, setb)
        drain(seta)
        compute_store(c0, seta)
        load_fire(c0 + 2, seta)
        drain(setb)
        compute_store(c0 + 1, setb)
        return carry

    lax.fori_loop(0, NLC // 2, pipe, 0)
    drain(seta)
    compute_store(NLC - 1, seta)
    drain_stores(seta)
    drain_stores(setb)


_stage_a = pl.kernel(
    _stage_a_body,
    out_type=tuple(jax.ShapeDtypeStruct((N_LINKS,), _f32) for _ in range(6)),
    mesh=_MESH,
    compiler_params=pltpu.CompilerParams(needs_layout_passes=False),
    scratch_types=(
        [pltpu.VMEM_SHARED((N_NODES,), _f32) for _ in range(3)]
        + 2 * ([pltpu.VMEM((CL,), _i32) for _ in range(2)]
               + [pltpu.VMEM((CL,), _f32) for _ in range(7)])
        + [pltpu.VMEM((CL,), _f32) for _ in range(12)]
        + [pltpu.VMEM((25_000,), _f32)]
        + [pltpu.SemaphoreType.DMA for _ in range(4)]
    ),
)


CNB = 400
NCHB = N_NODES // CNB
LANC = CNB * LPN


def _stage_pipelined_tbl(field, fsp, bounce_a, bounce_b, sems, base_st):
    csz = 5_000
    nst = (N_LINKS // NS) // csz
    dummy = field.at[pl.ds(0, csz)]

    def sl_of(j):
        return pl.ds(pl.multiple_of(base_st + j * csz, 8), csz)

    bufs = (bounce_a.at[pl.ds(0, csz)], bounce_b.at[pl.ds(0, csz)])
    pltpu.async_copy(field.at[sl_of(0)], bufs[0], sems[0])
    for j in range(nst):
        buf, sem = bufs[j % 2], sems[j % 2]
        pltpu.make_async_copy(dummy, buf, sem).wait()
        if j + 1 < nst:
            pltpu.async_copy(field.at[sl_of(j + 1)], bufs[(j + 1) % 2],
                             sems[(j + 1) % 2])
        pltpu.sync_copy(buf, fsp.at[sl_of(j)])


def _stage_b_body(gx, gy, lan, gxn, gyn,
                  fsp, lan_a, lan_b, g_a, g_b, o_a, o_b,
                  semg_a, semg_b, semlan_a, semlan_b):
    c = lax.axis_index("c")
    s = lax.axis_index("s")

    def run(field, out):
        _stage_pipelined_tbl(field, fsp, g_a, g_b, (semlan_a, semlan_b),
                             s * (N_LINKS // NS))
        plsc.subcore_barrier()
        iota16 = lax.iota(_i32, 16) * 16
        nib = (NCHB + NS - 1) // NS
        sets = ((lan_a, g_a, o_a, semg_a, semlan_a),
                (lan_b, g_b, o_b, semg_b, semlan_b))

        def chv(i):
            ch = s + i * NS
            return ch, jnp.where(ch < NCHB, ch, 0)

        def lan_fire(i, p):
            _, safe = chv(i)
            lan_v, _, _, _, semlan = sets[p]
            off = pl.ds(pl.multiple_of(safe * LANC, 8), LANC)
            pltpu.async_copy(lan.at[off], lan_v, semlan)

        def gath(p):
            lan_v, g_v, _, semg, semlan = sets[p]
            pltpu.make_async_copy(lan.at[pl.ds(0, LANC)], lan_v,
                                  semlan).wait()
            pltpu.async_copy(fsp.at[lan_v], g_v, semg)

        def red(i, p):
            ch, _ = chv(i)
            _, g_v, o_v, semg, _ = sets[p]
            pltpu.make_async_copy(field.at[pl.ds(0, LANC)], g_v, semg).wait()

            def body(ii, carry2):
                b = ii * (16 * LPN)
                acc = jnp.zeros((16,), _f32)
                for k in range(LPN):
                    acc = acc + plsc.load_gather(g_v, [iota16 + (b + k)])
                o_v[pl.ds(ii * 16, 16)] = acc * (1.0 / LPN)
                return carry2

            lax.fori_loop(0, CNB // 16, body, 0)

            @pl.when(ch < NCHB)
            def _():
                pltpu.sync_copy(
                    o_v, out.at[pl.ds(pl.multiple_of(ch * CNB, 8), CNB)])

        lan_fire(0, 0)
        gath(0)
        lan_fire(1, 1)

        def pipe(k, carry):
            c0 = k * 2
            gath(1)
            red(c0, 0)
            lan_fire(c0 + 2, 0)
            gath(0)
            red(c0 + 1, 1)
            lan_fire(c0 + 3, 1)
            return carry

        lax.fori_loop(0, nib // 2 - 1, pipe, 0)
        gath(1)
        red(nib - 2, 0)
        red(nib - 1, 1)

    @pl.when(c == 0)
    def _():
        run(gx, gxn)

    @pl.when(c == 1)
    def _():
        run(gy, gyn)


_stage_b = pl.kernel(
    _stage_b_body,
    out_type=tuple(jax.ShapeDtypeStruct((N_NODES,), _f32) for _ in range(2)),
    mesh=_MESH,
    compiler_params=pltpu.CompilerParams(needs_layout_passes=False),
    scratch_types=(
        [pltpu.VMEM_SHARED((N_LINKS,), _f32)]
        + [pltpu.VMEM((LANC,), _i32) for _ in range(2)]
        + [pltpu.VMEM((LANC,), _f32) for _ in range(2)]
        + [pltpu.VMEM((CNB,), _f32) for _ in range(2)]
        + [pltpu.SemaphoreType.DMA for _ in range(4)]
    ),
)


def _stage_c_body(nx, ny, tr, gxn, gyn, uidx, vel, cen, dif, gox, goy,
                  flux,
                  ps, gxs, gys,
                  uidx_a, vel_a, cen_a, dif_a, gox_a, goy_a,
                  up_a, ugx_a, ugy_a,
                  uidx_b, vel_b, cen_b, dif_b, gox_b, goy_b,
                  up_b, ugx_b, ugy_b,
                  flux_a, flux_b,
                  sx_v, sy_v, st_v, sgx_v, sgy_v,
                  sem_a, sem_b, sst_a, sst_b, slin_a, slin_b):
    s = lax.axis_index("s")

    @pl.when(s < 10)
    def _stage_tables():
        o = pl.multiple_of(s * 10_000, 8)
        slt = pl.ds(o, 10_000)
        pltpu.sync_copy(nx.at[slt], sx_v)
        pltpu.sync_copy(ny.at[slt], sy_v)
        pltpu.sync_copy(tr.at[slt], st_v)
        pltpu.sync_copy(gxn.at[slt], sgx_v)
        pltpu.sync_copy(gyn.at[slt], sgy_v)

        def pbody(j, carry):
            vs = pl.ds(j * 16, 16)
            st_v[vs] = (st_v[vs] + sx_v[vs] * sgx_v[vs]
                        + sy_v[vs] * sgy_v[vs])
            return carry

        lax.fori_loop(0, 10_000 // 16, pbody, 0)
        pltpu.sync_copy(st_v, ps.at[slt])
        pltpu.sync_copy(sgx_v, gxs.at[slt])
        pltpu.sync_copy(sgy_v, gys.at[slt])

    plsc.subcore_barrier()
    base = _wid() * LW

    seta = (uidx_a, vel_a, cen_a, dif_a, gox_a, goy_a, up_a, ugx_a, ugy_a,
            sem_a, flux_a, sst_a, slin_a)
    setb = (uidx_b, vel_b, cen_b, dif_b, gox_b, goy_b, up_b, ugx_b, ugy_b,
            sem_b, flux_b, sst_b, slin_b)

    def load_fire(ci, st):
        (uidx_v, vel_v, cen_v, dif_v, gox_v, goy_v, up_v, ugx_v, ugy_v,
         sem) = st[:10]
        slin = st[12]
        sl = pl.ds(pl.multiple_of(base + ci * CL, 8), CL)
        pltpu.sync_copy(uidx.at[sl], uidx_v)
        pltpu.async_copy(vel.at[sl], vel_v, slin)
        pltpu.async_copy(cen.at[sl], cen_v, slin)
        pltpu.async_copy(dif.at[sl], dif_v, slin)
        pltpu.async_copy(gox.at[sl], gox_v, slin)
        pltpu.async_copy(goy.at[sl], goy_v, slin)
        pltpu.async_copy(ps.at[uidx_v], up_v, sem)
        pltpu.async_copy(gxs.at[uidx_v], ugx_v, sem)
        pltpu.async_copy(gys.at[uidx_v], ugy_v, sem)

    def drain(st):
        sem = st[9]
        slin = st[12]
        dummy = nx.at[pl.ds(0, CL)]
        for dst in st[1:6]:
            pltpu.make_async_copy(dummy, dst, slin).wait()
        for dst in st[6:9]:
            pltpu.make_async_copy(dummy, dst, sem).wait()

    def compute_store(ci, st):
        _, vel_v, cen_v, dif_v, gox_v, goy_v, up_v, ugx_v, ugy_v = st[:9]
        flux_v, sst = st[10:12]
        sl = pl.ds(pl.multiple_of(base + ci * CL, 8), CL)

        @pl.when(ci >= 2)
        def _():
            pltpu.make_async_copy(nx.at[pl.ds(0, CL)], flux_v, sst).wait()

        def vec(j, carry2):
            vs = pl.ds(j * 16, 16)
            up = (up_v[vs] - gox_v[vs] * ugx_v[vs]
                  - goy_v[vs] * ugy_v[vs])
            ce = cen_v[vs]
            di = dif_v[vs]
            nz = di != 0.0
            den = jnp.where(nz, di, 1.0)
            r = jnp.where(nz, (ce - up) / den, 0.0)
            ar = jnp.abs(r)
            phi = (r + ar) / (1.0 + ar)
            flux_v[vs] = vel_v[vs] * (ce + 0.5 * phi * di)
            return carry2

        lax.fori_loop(0, CL // 16, vec, 0)
        pltpu.async_copy(flux_v, flux.at[sl], sst)

    load_fire(0, seta)

    def pipe(i, carry):
        c0 = i * 2
        load_fire(c0 + 1, setb)
        drain(seta)
        compute_store(c0, seta)
        load_fire(c0 + 2, seta)
        drain(setb)
        compute_store(c0 + 1, setb)
        return carry

    lax.fori_loop(0, NLC // 2, pipe, 0)
    drain(seta)
    compute_store(NLC - 1, seta)
    pltpu.make_async_copy(nx.at[pl.ds(0, CL)], flux_a, sst_a).wait()
    pltpu.make_async_copy(nx.at[pl.ds(0, CL)], flux_b, sst_b).wait()


_stage_c = pl.kernel(
    _stage_c_body,
    out_type=jax.ShapeDtypeStruct((N_LINKS,), _f32),
    mesh=_MESH,
    compiler_params=pltpu.CompilerParams(needs_layout_passes=False),
    scratch_types=(
        [pltpu.VMEM_SHARED((N_NODES,), _f32) for _ in range(3)]
        + 2 * ([pltpu.VMEM((CL,), _i32)]
               + [pltpu.VMEM((CL,), _f32) for _ in range(8)])
        + [pltpu.VMEM((CL,), _f32) for _ in range(2)]
        + [pltpu.VMEM((10_000,), _f32) for _ in range(5)]
        + [pltpu.SemaphoreType.DMA for _ in range(6)]
    ),
)


def _stage_d_body(flux, lan, tr, area, dt16, out,
                  fsp, lan_a, lan_b, g_a, g_b, tr_a, tr_b, ar_a, ar_b,
                  o_a, o_b, dt_v,
                  semg_a, semg_b, semlan_a, semlan_b):
    s = lax.axis_index("s")
    w = _wid()

    _stage_pipelined_tbl(flux, fsp, g_a, g_b, (semlan_a, semlan_b),
                         s * (N_LINKS // NS))
    pltpu.sync_copy(dt16, dt_v)
    plsc.subcore_barrier()
    dtv = dt_v[...]
    iota16 = lax.iota(_i32, 16) * 16
    nid = (NCHB + NW - 1) // NW
    sets = ((lan_a, g_a, tr_a, ar_a, o_a, semg_a, semlan_a),
            (lan_b, g_b, tr_b, ar_b, o_b, semg_b, semlan_b))

    def chv(i):
        ch = w + i * NW
        return ch, jnp.where(ch < NCHB, ch, 0)

    def lan_fire(i, p):
        _, safe = chv(i)
        lan_v, _, tr_v, ar_v, _, _, semlan = sets[p]
        off = pl.ds(pl.multiple_of(safe * LANC, 8), LANC)
        noff = pl.ds(pl.multiple_of(safe * CNB, 8), CNB)
        pltpu.async_copy(lan.at[off], lan_v, semlan)
        pltpu.async_copy(tr.at[noff], tr_v, semlan)
        pltpu.async_copy(area.at[noff], ar_v, semlan)

    def gath(p):
        lan_v, g_v, tr_v, ar_v, _, semg, semlan = sets[p]
        pltpu.make_async_copy(lan.at[pl.ds(0, LANC)], lan_v, semlan).wait()
        pltpu.make_async_copy(tr.at[pl.ds(0, CNB)], tr_v, semlan).wait()
        pltpu.make_async_copy(tr.at[pl.ds(0, CNB)], ar_v, semlan).wait()
        pltpu.async_copy(fsp.at[lan_v], g_v, semg)

    def red(i, p):
        ch, _ = chv(i)
        _, g_v, tr_v, ar_v, o_v, semg, _ = sets[p]
        pltpu.make_async_copy(flux.at[pl.ds(0, LANC)], g_v, semg).wait()

        def body(ii, carry2):
            b = ii * (16 * LPN)
            acc = jnp.zeros((16,), _f32)
            for k in range(LPN):
                acc = acc + plsc.load_gather(g_v, [iota16 + (b + k)])
            vs = pl.ds(ii * 16, 16)
            a = ar_v[vs]
            nz = a != 0.0
            asafe = jnp.where(nz, a, 1.0)
            div = jnp.where(nz, acc / asafe, 0.0)
            o_v[vs] = tr_v[vs] + dtv * div
            return carry2

        lax.fori_loop(0, CNB // 16, body, 0)

        @pl.when(ch < NCHB)
        def _():
            pltpu.sync_copy(
                o_v, out.at[pl.ds(pl.multiple_of(ch * CNB, 8), CNB)])

    lan_fire(0, 0)
    gath(0)
    lan_fire(1, 1)

    def pipe(k, carry):
        c0 = k * 2
        gath(1)
        red(c0, 0)
        lan_fire(c0 + 2, 0)
        gath(0)
        red(c0 + 1, 1)
        lan_fire(c0 + 3, 1)
        return carry

    lax.fori_loop(0, nid // 2 - 1, pipe, 0)
    gath(1)
    red(nid - 2, 0)
    red(nid - 1, 1)


_stage_d = pl.kernel(
    _stage_d_body,
    out_type=jax.ShapeDtypeStruct((N_NODES,), _f32),
    mesh=_MESH,
    compiler_params=pltpu.CompilerParams(needs_layout_passes=False),
    scratch_types=(
        [pltpu.VMEM_SHARED((N_LINKS,), _f32)]
        + [pltpu.VMEM((LANC,), _i32) for _ in range(2)]
        + [pltpu.VMEM((LANC,), _f32) for _ in range(2)]
        + [pltpu.VMEM((CNB,), _f32) for _ in range(6)]
        + [pltpu.VMEM((16,), _f32)]
        + [pltpu.SemaphoreType.DMA for _ in range(4)]
    ),
)


def kernel(velocity, tracer, node_x, node_y, cell_area_at_node, dt,
           node_at_link_head, node_at_link_tail, links_at_node,
           upwind_real_idx):
    hidx = node_at_link_head.astype(_i32)
    tidx = node_at_link_tail.astype(_i32)
    uidx = upwind_real_idx.astype(_i32)
    lan_flat = links_at_node.astype(_i32).reshape(-1)
    dt16 = jnp.broadcast_to(dt.astype(_f32), (16,))

    gx, gy, cen, dif, gox, goy = _stage_a(
        node_x, node_y, tracer, hidx, tidx, velocity)
    gxn, gyn = _stage_b(gx, gy, lan_flat)
    flux = _stage_c(node_x, node_y, tracer, gxn, gyn, uidx, velocity,
                    cen, dif, gox, goy)
    return _stage_d(flux, lan_flat, tracer, cell_area_at_node, dt16)


# speedup vs baseline: 308.0903x; 1.0960x over previous
import functools

import jax
import jax.numpy as jnp
from jax import lax
from jax.experimental import pallas as pl
from jax.experimental.pallas import tpu as pltpu
from jax.experimental.pallas import tpu_sc as plsc

N_NODES = 100_000
N_LINKS = 1_600_000
LPN = 16

NC = 2
NS = 16
NW = NC * NS

LW = N_LINKS // NW
CL = 2_000
NLC = LW // CL

CN = 800
NCH = N_NODES // CN

_MESH = plsc.VectorSubcoreMesh(
    core_axis_name="c", subcore_axis_name="s", num_cores=NC, num_subcores=NS)

_f32 = jnp.float32
_i32 = jnp.int32


def _wid():
    return lax.axis_index("s") * NC + lax.axis_index("c")


def _hbm_to_spmem(hbm_ref, sp_ref, vtmp, base, nchunks, csz=None):
    if csz is None:
        csz = vtmp.shape[0]

    def body(i, carry):
        o = pl.multiple_of(base + i * csz, 8)
        sl = pl.ds(o, csz)
        pltpu.sync_copy(hbm_ref.at[sl], vtmp.at[pl.ds(0, csz)])
        pltpu.sync_copy(vtmp.at[pl.ds(0, csz)], sp_ref.at[sl])
        return carry

    lax.fori_loop(0, nchunks, body, 0)


def _stage_a_body(nx, ny, tr, hidx, tidx, vel,
                  gxy, cen, dif, gox, goy,
                  xs, ys, ts,
                  hidx_a, tidx_a, vel_a, hx_a, hy_a, htr_a, tx_a, ty_a, ttr_a,
                  hidx_b, tidx_b, vel_b, hx_b, hy_b, htr_b, tx_b, ty_b, ttr_b,
                  gxy_a, cen_a, dif_a, gox_a, goy_a,
                  gxy_b, cen_b, dif_b, gox_b, goy_b,
                  stg_v,
                  sem_a, sem_b, sst_a, sst_b):
    s = lax.axis_index("s")
    base = _wid() * LW

    @pl.when(s < 4)
    def _stage_tables():
        base_t = s * 25_000
        _hbm_to_spmem(nx, xs, stg_v, base_t, 1)
        _hbm_to_spmem(ny, ys, stg_v, base_t, 1)
        _hbm_to_spmem(tr, ts, stg_v, base_t, 1)

    plsc.subcore_barrier()

    seta = (hidx_a, tidx_a, vel_a, hx_a, hy_a, htr_a, tx_a, ty_a, ttr_a,
            sem_a, gxy_a, cen_a, dif_a, gox_a, goy_a, sst_a)
    setb = (hidx_b, tidx_b, vel_b, hx_b, hy_b, htr_b, tx_b, ty_b, ttr_b,
            sem_b, gxy_b, cen_b, dif_b, gox_b, goy_b, sst_b)

    def load_fire(ci, st):
        hidx_v, tidx_v, vel_v, hx_v, hy_v, htr_v, tx_v, ty_v, ttr_v, sem = \
            st[:10]
        sl = pl.ds(pl.multiple_of(base + ci * CL, 8), CL)
        pltpu.sync_copy(hidx.at[sl], hidx_v)
        pltpu.sync_copy(tidx.at[sl], tidx_v)
        pltpu.sync_copy(vel.at[sl], vel_v)
        pltpu.async_copy(xs.at[hidx_v], hx_v, sem)
        pltpu.async_copy(ys.at[hidx_v], hy_v, sem)
        pltpu.async_copy(ts.at[hidx_v], htr_v, sem)
        pltpu.async_copy(xs.at[tidx_v], tx_v, sem)
        pltpu.async_copy(ys.at[tidx_v], ty_v, sem)
        pltpu.async_copy(ts.at[tidx_v], ttr_v, sem)

    def drain(st):
        sem = st[9]
        dummy = nx.at[pl.ds(0, CL)]
        for dst in st[3:9]:
            pltpu.make_async_copy(dummy, dst, sem).wait()

    def drain_stores(st):
        sst = st[15]
        dummy = nx.at[pl.ds(0, CL)]
        for dst in st[10:15]:
            pltpu.make_async_copy(dummy, dst, sst).wait()

    def compute_store(ci, st):
        _, _, vel_v, hx_v, hy_v, htr_v, tx_v, ty_v, ttr_v = st[:9]
        gxy_v, cen_v, dif_v, gox_v, goy_v, sst = st[10:]
        sl = pl.ds(pl.multiple_of(base + ci * CL, 8), CL)

        @pl.when(ci >= 2)
        def _():
            drain_stores(st)

        def vec(j, carry2):
            vs = pl.ds(j * 16, 16)
            hx = hx_v[vs]; hy = hy_v[vs]; htr = htr_v[vs]
            tx = tx_v[vs]; ty = ty_v[vs]; ttr = ttr_v[vs]
            v = vel_v[vs]
            dx = hx - tx
            dy = hy - ty
            l2 = jnp.maximum(dx * dx + dy * dy, 1e-18)
            dtr = (htr - ttr) / l2
            gxi = lax.bitcast_convert_type(dtr * dx, _i32) + 0x8000
            gyi = lax.bitcast_convert_type(dtr * dy, _i32) + 0x8000
            gxy_v[vs] = ((gxi & jnp.int32(-65536))
                         | (lax.shift_right_logical(gyi, 16) & 0xFFFF))
            vpos = v >= 0.0
            cen_v[vs] = jnp.where(vpos, ttr, htr)
            dif_v[vs] = jnp.where(vpos, htr - ttr, ttr - htr)
            gox_v[vs] = jnp.where(vpos, 2.0 * hx - tx, 2.0 * tx - hx)
            goy_v[vs] = jnp.where(vpos, 2.0 * hy - ty, 2.0 * ty - hy)
            return carry2

        lax.fori_loop(0, CL // 16, vec, 0)
        pltpu.async_copy(gxy_v, gxy.at[sl], sst)
        pltpu.async_copy(cen_v, cen.at[sl], sst)
        pltpu.async_copy(dif_v, dif.at[sl], sst)
        pltpu.async_copy(gox_v, gox.at[sl], sst)
        pltpu.async_copy(goy_v, goy.at[sl], sst)

    load_fire(0, seta)

    def pipe(i, carry):
        c0 = i * 2
        load_fire(c0 + 1, setb)
        drain(seta)
        compute_store(c0, seta)
        load_fire(c0 + 2, seta)
        drain(setb)
        compute_store(c0 + 1, setb)
        return carry

    lax.fori_loop(0, NLC // 2, pipe, 0)
    drain(seta)
    compute_store(NLC - 1, seta)
    drain_stores(seta)
    drain_stores(setb)


_stage_a = pl.kernel(
    _stage_a_body,
    out_type=((jax.ShapeDtypeStruct((N_LINKS,), _i32),)
              + tuple(jax.ShapeDtypeStruct((N_LINKS,), _f32)
                      for _ in range(4))),
    mesh=_MESH,
    compiler_params=pltpu.CompilerParams(needs_layout_passes=False),
    scratch_types=(
        [pltpu.VMEM_SHARED((N_NODES,), _f32) for _ in range(3)]
        + 2 * ([pltpu.VMEM((CL,), _i32) for _ in range(2)]
               + [pltpu.VMEM((CL,), _f32) for _ in range(7)])
        + 2 * ([pltpu.VMEM((CL,), _i32)]
               + [pltpu.VMEM((CL,), _f32) for _ in range(4)])
        + [pltpu.VMEM((25_000,), _f32)]
        + [pltpu.SemaphoreType.DMA for _ in range(4)]
    ),
)


CNB = 400
NCHB = N_NODES // CNB
LANC = CNB * LPN


def _stage_pipelined_tbl(field, fsp, bounce_a, bounce_b, sems, base_st):
    csz = 5_000
    nst = (N_LINKS // NS) // csz
    dummy = field.at[pl.ds(0, csz)]

    def sl_of(j):
        return pl.ds(pl.multiple_of(base_st + j * csz, 8), csz)

    bufs = (bounce_a.at[pl.ds(0, csz)], bounce_b.at[pl.ds(0, csz)])
    pltpu.async_copy(field.at[sl_of(0)], bufs[0], sems[0])
    for j in range(nst):
        buf, sem = bufs[j % 2], sems[j % 2]
        pltpu.make_async_copy(dummy, buf, sem).wait()
        if j + 1 < nst:
            pltpu.async_copy(field.at[sl_of(j + 1)], bufs[(j + 1) % 2],
                             sems[(j + 1) % 2])
        pltpu.sync_copy(buf, fsp.at[sl_of(j)])


def _stage_b_body(gxy, lan, gxn, gyn,
                  fsp, lan_a, lan_b, g_a, g_b, ox_a, ox_b, oy_a, oy_b,
                  semg_a, semg_b, semlan_a, semlan_b):
    s = lax.axis_index("s")
    w = _wid()

    _stage_pipelined_tbl(gxy, fsp, g_a, g_b, (semlan_a, semlan_b),
                         s * (N_LINKS // NS))
    plsc.subcore_barrier()
    iota16 = lax.iota(_i32, 16) * 16
    nib = (NCHB + NW - 1) // NW
    sets = ((lan_a, g_a, ox_a, oy_a, semg_a, semlan_a),
            (lan_b, g_b, ox_b, oy_b, semg_b, semlan_b))

    def chv(i):
        ch = w + i * NW
        return ch, jnp.where(ch < NCHB, ch, 0)

    def lan_fire(i, p):
        _, safe = chv(i)
        lan_v, _, _, _, _, semlan = sets[p]
        off = pl.ds(pl.multiple_of(safe * LANC, 8), LANC)
        pltpu.async_copy(lan.at[off], lan_v, semlan)

    def gath(p):
        lan_v, g_v, _, _, semg, semlan = sets[p]
        pltpu.make_async_copy(lan.at[pl.ds(0, LANC)], lan_v, semlan).wait()
        pltpu.async_copy(fsp.at[lan_v], g_v, semg)

    def red(i, p):
        ch, _ = chv(i)
        _, g_v, ox_v, oy_v, semg, _ = sets[p]
        pltpu.make_async_copy(lan.at[pl.ds(0, LANC)], g_v, semg).wait()

        def body(ii, carry2):
            b = ii * (16 * LPN)
            accx = jnp.zeros((16,), _f32)
            accy = jnp.zeros((16,), _f32)
            for k in range(LPN):
                wv = plsc.load_gather(g_v, [iota16 + (b + k)])
                accx = accx + lax.bitcast_convert_type(
                    wv & jnp.int32(-65536), _f32)
                accy = accy + lax.bitcast_convert_type(
                    lax.shift_left(wv, 16), _f32)
            vs = pl.ds(ii * 16, 16)
            ox_v[vs] = accx * (1.0 / LPN)
            oy_v[vs] = accy * (1.0 / LPN)
            return carry2

        lax.fori_loop(0, CNB // 16, body, 0)

        @pl.when(ch < NCHB)
        def _():
            noff = pl.ds(pl.multiple_of(ch * CNB, 8), CNB)
            pltpu.sync_copy(ox_v, gxn.at[noff])
            pltpu.sync_copy(oy_v, gyn.at[noff])

    lan_fire(0, 0)
    gath(0)
    lan_fire(1, 1)

    def pipe(k, carry):
        c0 = k * 2
        gath(1)
        red(c0, 0)
        lan_fire(c0 + 2, 0)
        gath(0)
        red(c0 + 1, 1)
        lan_fire(c0 + 3, 1)
        return carry

    lax.fori_loop(0, nib // 2 - 1, pipe, 0)
    gath(1)
    red(nib - 2, 0)
    red(nib - 1, 1)


_stage_b = pl.kernel(
    _stage_b_body,
    out_type=tuple(jax.ShapeDtypeStruct((N_NODES,), _f32) for _ in range(2)),
    mesh=_MESH,
    compiler_params=pltpu.CompilerParams(needs_layout_passes=False),
    scratch_types=(
        [pltpu.VMEM_SHARED((N_LINKS,), _i32)]
        + [pltpu.VMEM((LANC,), _i32) for _ in range(4)]
        + [pltpu.VMEM((CNB,), _f32) for _ in range(4)]
        + [pltpu.SemaphoreType.DMA for _ in range(4)]
    ),
)


def _stage_c_body(nx, ny, tr, gxn, gyn, uidx, vel, cen, dif, gox, goy,
                  flux,
                  ps, gps,
                  uidx_a, vel_a, cen_a, dif_a, gox_a, goy_a,
                  up_a, ug_a,
                  uidx_b, vel_b, cen_b, dif_b, gox_b, goy_b,
                  up_b, ug_b,
                  flux_a, flux_b,
                  sx_v, sy_v, st_v, sgx_v, sgy_v, sgp_v,
                  sem_a, sem_b, sst_a, sst_b, slin_a, slin_b):
    s = lax.axis_index("s")

    @pl.when(s < 10)
    def _stage_tables():
        o = pl.multiple_of(s * 10_000, 8)
        slt = pl.ds(o, 10_000)
        pltpu.sync_copy(nx.at[slt], sx_v)
        pltpu.sync_copy(ny.at[slt], sy_v)
        pltpu.sync_copy(tr.at[slt], st_v)
        pltpu.sync_copy(gxn.at[slt], sgx_v)
        pltpu.sync_copy(gyn.at[slt], sgy_v)

        def pbody(j, carry):
            vs = pl.ds(j * 16, 16)
            gxv = sgx_v[vs]
            gyv = sgy_v[vs]
            st_v[vs] = st_v[vs] + sx_v[vs] * gxv + sy_v[vs] * gyv
            gxi = lax.bitcast_convert_type(gxv, _i32) + 0x8000
            gyi = lax.bitcast_convert_type(gyv, _i32) + 0x8000
            sgp_v[vs] = ((gxi & jnp.int32(-65536))
                         | (lax.shift_right_logical(gyi, 16) & 0xFFFF))
            return carry

        lax.fori_loop(0, 10_000 // 16, pbody, 0)
        pltpu.sync_copy(st_v, ps.at[slt])
        pltpu.sync_copy(sgp_v, gps.at[slt])

    plsc.subcore_barrier()
    base = _wid() * LW

    seta = (uidx_a, vel_a, cen_a, dif_a, gox_a, goy_a, up_a, ug_a,
            sem_a, flux_a, sst_a, slin_a)
    setb = (uidx_b, vel_b, cen_b, dif_b, gox_b, goy_b, up_b, ug_b,
            sem_b, flux_b, sst_b, slin_b)

    def load_fire(ci, st):
        (uidx_v, vel_v, cen_v, dif_v, gox_v, goy_v, up_v, ug_v,
         sem) = st[:9]
        slin = st[11]
        sl = pl.ds(pl.multiple_of(base + ci * CL, 8), CL)
        pltpu.sync_copy(uidx.at[sl], uidx_v)
        pltpu.async_copy(vel.at[sl], vel_v, slin)
        pltpu.async_copy(cen.at[sl], cen_v, slin)
        pltpu.async_copy(dif.at[sl], dif_v, slin)
        pltpu.async_copy(gox.at[sl], gox_v, slin)
        pltpu.async_copy(goy.at[sl], goy_v, slin)
        pltpu.async_copy(ps.at[uidx_v], up_v, sem)
        pltpu.async_copy(gps.at[uidx_v], ug_v, sem)

    def drain(st):
        sem = st[8]
        slin = st[11]
        dummy = nx.at[pl.ds(0, CL)]
        for dst in st[1:6]:
            pltpu.make_async_copy(dummy, dst, slin).wait()
        for dst in st[6:8]:
            pltpu.make_async_copy(dummy, dst, sem).wait()

    def compute_store(ci, st):
        _, vel_v, cen_v, dif_v, gox_v, goy_v, up_v, ug_v = st[:8]
        flux_v, sst = st[9:11]
        sl = pl.ds(pl.multiple_of(base + ci * CL, 8), CL)

        @pl.when(ci >= 2)
        def _():
            pltpu.make_async_copy(nx.at[pl.ds(0, CL)], flux_v, sst).wait()

        def vec(j, carry2):
            vs = pl.ds(j * 16, 16)
            wv = ug_v[vs]
            ugx = lax.bitcast_convert_type(wv & jnp.int32(-65536), _f32)
            ugy = lax.bitcast_convert_type(lax.shift_left(wv, 16), _f32)
            up = (up_v[vs] - gox_v[vs] * ugx - goy_v[vs] * ugy)
            ce = cen_v[vs]
            di = dif_v[vs]
            nz = di != 0.0
            den = jnp.where(nz, di, 1.0)
            r = jnp.where(nz, (ce - up) / den, 0.0)
            ar = jnp.abs(r)
            phi = (r + ar) / (1.0 + ar)
            flux_v[vs] = vel_v[vs] * (ce + 0.5 * phi * di)
            return carry2

        lax.fori_loop(0, CL // 16, vec, 0)
        pltpu.async_copy(flux_v, flux.at[sl], sst)

    load_fire(0, seta)

    def pipe(i, carry):
        c0 = i * 2
        load_fire(c0 + 1, setb)
        drain(seta)
        compute_store(c0, seta)
        load_fire(c0 + 2, seta)
        drain(setb)
        compute_store(c0 + 1, setb)
        return carry

    lax.fori_loop(0, NLC // 2, pipe, 0)
    drain(seta)
    compute_store(NLC - 1, seta)
    pltpu.make_async_copy(nx.at[pl.ds(0, CL)], flux_a, sst_a).wait()
    pltpu.make_async_copy(nx.at[pl.ds(0, CL)], flux_b, sst_b).wait()


_stage_c = pl.kernel(
    _stage_c_body,
    out_type=jax.ShapeDtypeStruct((N_LINKS,), _f32),
    mesh=_MESH,
    compiler_params=pltpu.CompilerParams(needs_layout_passes=False),
    scratch_types=(
        [pltpu.VMEM_SHARED((N_NODES,), _f32), pltpu.VMEM_SHARED((N_NODES,), _i32)]
        + 2 * ([pltpu.VMEM((CL,), _i32)]
               + [pltpu.VMEM((CL,), _f32) for _ in range(6)]
               + [pltpu.VMEM((CL,), _i32)])
        + [pltpu.VMEM((CL,), _f32) for _ in range(2)]
        + [pltpu.VMEM((10_000,), _f32) for _ in range(5)]
        + [pltpu.VMEM((10_000,), _i32)]
        + [pltpu.SemaphoreType.DMA for _ in range(6)]
    ),
)


def _stage_d_body(flux, lan, tr, area, dt16, out,
                  fsp, lan_a, lan_b, g_a, g_b, tr_a, tr_b, ar_a, ar_b,
                  o_a, o_b, dt_v,
                  semg_a, semg_b, semlan_a, semlan_b):
    s = lax.axis_index("s")
    w = _wid()

    _stage_pipelined_tbl(flux, fsp, g_a, g_b, (semlan_a, semlan_b),
                         s * (N_LINKS // NS))
    pltpu.sync_copy(dt16, dt_v)
    plsc.subcore_barrier()
    dtv = dt_v[...]
    iota16 = lax.iota(_i32, 16) * 16
    nid = (NCHB + NW - 1) // NW
    sets = ((lan_a, g_a, tr_a, ar_a, o_a, semg_a, semlan_a),
            (lan_b, g_b, tr_b, ar_b, o_b, semg_b, semlan_b))

    def chv(i):
        ch = w + i * NW
        return ch, jnp.where(ch < NCHB, ch, 0)

    def lan_fire(i, p):
        _, safe = chv(i)
        lan_v, _, tr_v, ar_v, _, _, semlan = sets[p]
        off = pl.ds(pl.multiple_of(safe * LANC, 8), LANC)
        noff = pl.ds(pl.multiple_of(safe * CNB, 8), CNB)
        pltpu.async_copy(lan.at[off], lan_v, semlan)
        pltpu.async_copy(tr.at[noff], tr_v, semlan)
        pltpu.async_copy(area.at[noff], ar_v, semlan)

    def gath(p):
        lan_v, g_v, tr_v, ar_v, _, semg, semlan = sets[p]
        pltpu.make_async_copy(lan.at[pl.ds(0, LANC)], lan_v, semlan).wait()
        pltpu.make_async_copy(tr.at[pl.ds(0, CNB)], tr_v, semlan).wait()
        pltpu.make_async_copy(tr.at[pl.ds(0, CNB)], ar_v, semlan).wait()
        pltpu.async_copy(fsp.at[lan_v], g_v, semg)

    def red(i, p):
        ch, _ = chv(i)
        _, g_v, tr_v, ar_v, o_v, semg, _ = sets[p]
        pltpu.make_async_copy(flux.at[pl.ds(0, LANC)], g_v, semg).wait()

        def body(ii, carry2):
            b = ii * (16 * LPN)
            acc = jnp.zeros((16,), _f32)
            for k in range(LPN):
                acc = acc + plsc.load_gather(g_v, [iota16 + (b + k)])
            vs = pl.ds(ii * 16, 16)
            a = ar_v[vs]
            nz = a != 0.0
            asafe = jnp.where(nz, a, 1.0)
            div = jnp.where(nz, acc / asafe, 0.0)
            o_v[vs] = tr_v[vs] + dtv * div
            return carry2

        lax.fori_loop(0, CNB // 16, body, 0)

        @pl.when(ch < NCHB)
        def _():
            pltpu.sync_copy(
                o_v, out.at[pl.ds(pl.multiple_of(ch * CNB, 8), CNB)])

    lan_fire(0, 0)
    gath(0)
    lan_fire(1, 1)

    def pipe(k, carry):
        c0 = k * 2
        gath(1)
        red(c0, 0)
        lan_fire(c0 + 2, 0)
        gath(0)
        red(c0 + 1, 1)
        lan_fire(c0 + 3, 1)
        return carry

    lax.fori_loop(0, nid // 2 - 1, pipe, 0)
    gath(1)
    red(nid - 2, 0)
    red(nid - 1, 1)


_stage_d = pl.kernel(
    _stage_d_body,
    out_type=jax.ShapeDtypeStruct((N_NODES,), _f32),
    mesh=_MESH,
    compiler_params=pltpu.CompilerParams(needs_layout_passes=False),
    scratch_types=(
        [pltpu.VMEM_SHARED((N_LINKS,), _f32)]
        + [pltpu.VMEM((LANC,), _i32) for _ in range(2)]
        + [pltpu.VMEM((LANC,), _f32) for _ in range(2)]
        + [pltpu.VMEM((CNB,), _f32) for _ in range(6)]
        + [pltpu.VMEM((16,), _f32)]
        + [pltpu.SemaphoreType.DMA for _ in range(4)]
    ),
)


def kernel(velocity, tracer, node_x, node_y, cell_area_at_node, dt,
           node_at_link_head, node_at_link_tail, links_at_node,
           upwind_real_idx):
    hidx = node_at_link_head.astype(_i32)
    tidx = node_at_link_tail.astype(_i32)
    uidx = upwind_real_idx.astype(_i32)
    lan_flat = links_at_node.astype(_i32).reshape(-1)
    dt16 = jnp.broadcast_to(dt.astype(_f32), (16,))

    gxy, cen, dif, gox, goy = _stage_a(
        node_x, node_y, tracer, hidx, tidx, velocity)
    gxn, gyn = _stage_b(gxy, lan_flat)
    flux = _stage_c(node_x, node_y, tracer, gxn, gyn, uidx, velocity,
                    cen, dif, gox, goy)
    return _stage_d(flux, lan_flat, tracer, cell_area_at_node, dt16)


# speedup vs baseline: 312.4978x; 1.0143x over previous
import functools

import jax
import jax.numpy as jnp
from jax import lax
from jax.experimental import pallas as pl
from jax.experimental.pallas import tpu as pltpu
from jax.experimental.pallas import tpu_sc as plsc

N_NODES = 100_000
N_LINKS = 1_600_000
LPN = 16

NC = 2
NS = 16
NW = NC * NS

LW = N_LINKS // NW
CL = 2_000
NLC = LW // CL

CN = 800
NCH = N_NODES // CN

_MESH = plsc.VectorSubcoreMesh(
    core_axis_name="c", subcore_axis_name="s", num_cores=NC, num_subcores=NS)

_f32 = jnp.float32
_i32 = jnp.int32


def _wid():
    return lax.axis_index("s") * NC + lax.axis_index("c")


def _hbm_to_spmem(hbm_ref, sp_ref, vtmp, base, nchunks, csz=None):
    if csz is None:
        csz = vtmp.shape[0]

    def body(i, carry):
        o = pl.multiple_of(base + i * csz, 8)
        sl = pl.ds(o, csz)
        pltpu.sync_copy(hbm_ref.at[sl], vtmp.at[pl.ds(0, csz)])
        pltpu.sync_copy(vtmp.at[pl.ds(0, csz)], sp_ref.at[sl])
        return carry

    lax.fori_loop(0, nchunks, body, 0)


def _stage_a_body(nx, ny, tr, hidx, tidx, vel,
                  gxy, cen, dif, gox, goy,
                  xs, ys, ts,
                  hidx_a, tidx_a, vel_a, hx_a, hy_a, htr_a, tx_a, ty_a, ttr_a,
                  hidx_b, tidx_b, vel_b, hx_b, hy_b, htr_b, tx_b, ty_b, ttr_b,
                  gxy_a, cen_a, dif_a, gox_a, goy_a,
                  gxy_b, cen_b, dif_b, gox_b, goy_b,
                  stg_v,
                  sem_a, sem_b, sst_a, sst_b):
    s = lax.axis_index("s")
    base = _wid() * LW

    @pl.when(s < 10)
    def _stage_tables():
        o = pl.multiple_of(s * 10_000, 8)
        slt = pl.ds(o, 10_000)
        bufs = (stg_v.at[pl.ds(0, 10_000)], stg_v.at[pl.ds(10_000, 10_000)])
        srcs = (nx, ny, tr)
        dsts = (xs, ys, ts)
        pltpu.async_copy(srcs[0].at[slt], bufs[0], sem_a)
        for j in range(3):
            buf = bufs[j % 2]
            pltpu.make_async_copy(srcs[j].at[slt], buf, sem_a).wait()
            if j + 1 < 3:
                pltpu.async_copy(srcs[j + 1].at[slt], bufs[(j + 1) % 2],
                                 sem_a)
            pltpu.sync_copy(buf, dsts[j].at[slt])

    plsc.subcore_barrier()

    seta = (hidx_a, tidx_a, vel_a, hx_a, hy_a, htr_a, tx_a, ty_a, ttr_a,
            sem_a, gxy_a, cen_a, dif_a, gox_a, goy_a, sst_a)
    setb = (hidx_b, tidx_b, vel_b, hx_b, hy_b, htr_b, tx_b, ty_b, ttr_b,
            sem_b, gxy_b, cen_b, dif_b, gox_b, goy_b, sst_b)

    def load_fire(ci, st):
        hidx_v, tidx_v, vel_v, hx_v, hy_v, htr_v, tx_v, ty_v, ttr_v, sem = \
            st[:10]
        sl = pl.ds(pl.multiple_of(base + ci * CL, 8), CL)
        pltpu.sync_copy(hidx.at[sl], hidx_v)
        pltpu.sync_copy(tidx.at[sl], tidx_v)
        pltpu.sync_copy(vel.at[sl], vel_v)
        pltpu.async_copy(xs.at[hidx_v], hx_v, sem)
        pltpu.async_copy(ys.at[hidx_v], hy_v, sem)
        pltpu.async_copy(ts.at[hidx_v], htr_v, sem)
        pltpu.async_copy(xs.at[tidx_v], tx_v, sem)
        pltpu.async_copy(ys.at[tidx_v], ty_v, sem)
        pltpu.async_copy(ts.at[tidx_v], ttr_v, sem)

    def drain(st):
        sem = st[9]
        dummy = nx.at[pl.ds(0, CL)]
        for dst in st[3:9]:
            pltpu.make_async_copy(dummy, dst, sem).wait()

    def drain_stores(st):
        sst = st[15]
        dummy = nx.at[pl.ds(0, CL)]
        for dst in st[10:15]:
            pltpu.make_async_copy(dummy, dst, sst).wait()

    def compute_store(ci, st):
        _, _, vel_v, hx_v, hy_v, htr_v, tx_v, ty_v, ttr_v = st[:9]
        gxy_v, cen_v, dif_v, gox_v, goy_v, sst = st[10:]
        sl = pl.ds(pl.multiple_of(base + ci * CL, 8), CL)

        @pl.when(ci >= 2)
        def _():
            drain_stores(st)

        def vec(j, carry2):
            vs = pl.ds(j * 16, 16)
            hx = hx_v[vs]; hy = hy_v[vs]; htr = htr_v[vs]
            tx = tx_v[vs]; ty = ty_v[vs]; ttr = ttr_v[vs]
            v = vel_v[vs]
            dx = hx - tx
            dy = hy - ty
            l2 = jnp.maximum(dx * dx + dy * dy, 1e-18)
            dtr = (htr - ttr) / l2
            gxi = lax.bitcast_convert_type(dtr * dx, _i32) + 0x8000
            gyi = lax.bitcast_convert_type(dtr * dy, _i32) + 0x8000
            gxy_v[vs] = ((gxi & jnp.int32(-65536))
                         | (lax.shift_right_logical(gyi, 16) & 0xFFFF))
            vpos = v >= 0.0
            cen_v[vs] = jnp.where(vpos, ttr, htr)
            dif_v[vs] = jnp.where(vpos, htr - ttr, ttr - htr)
            gox_v[vs] = jnp.where(vpos, 2.0 * hx - tx, 2.0 * tx - hx)
            goy_v[vs] = jnp.where(vpos, 2.0 * hy - ty, 2.0 * ty - hy)
            return carry2

        lax.fori_loop(0, CL // 16, vec, 0)
        pltpu.async_copy(gxy_v, gxy.at[sl], sst)
        pltpu.async_copy(cen_v, cen.at[sl], sst)
        pltpu.async_copy(dif_v, dif.at[sl], sst)
        pltpu.async_copy(gox_v, gox.at[sl], sst)
        pltpu.async_copy(goy_v, goy.at[sl], sst)

    load_fire(0, seta)

    def pipe(i, carry):
        c0 = i * 2
        load_fire(c0 + 1, setb)
        drain(seta)
        compute_store(c0, seta)
        load_fire(c0 + 2, seta)
        drain(setb)
        compute_store(c0 + 1, setb)
        return carry

    lax.fori_loop(0, NLC // 2, pipe, 0)
    drain(seta)
    compute_store(NLC - 1, seta)
    drain_stores(seta)
    drain_stores(setb)


_stage_a = pl.kernel(
    _stage_a_body,
    out_type=((jax.ShapeDtypeStruct((N_LINKS,), _i32),)
              + tuple(jax.ShapeDtypeStruct((N_LINKS,), _f32)
                      for _ in range(4))),
    mesh=_MESH,
    compiler_params=pltpu.CompilerParams(needs_layout_passes=False),
    scratch_types=(
        [pltpu.VMEM_SHARED((N_NODES,), _f32) for _ in range(3)]
        + 2 * ([pltpu.VMEM((CL,), _i32) for _ in range(2)]
               + [pltpu.VMEM((CL,), _f32) for _ in range(7)])
        + 2 * ([pltpu.VMEM((CL,), _i32)]
               + [pltpu.VMEM((CL,), _f32) for _ in range(4)])
        + [pltpu.VMEM((20_000,), _f32)]
        + [pltpu.SemaphoreType.DMA for _ in range(4)]
    ),
)


CNB = 400
NCHB = N_NODES // CNB
LANC = CNB * LPN


def _stage_pipelined_tbl(field, fsp, bounce_a, bounce_b, sems, base_st):
    csz = 5_000
    nst = (N_LINKS // NS) // csz
    dummy = field.at[pl.ds(0, csz)]

    def sl_of(j):
        return pl.ds(pl.multiple_of(base_st + j * csz, 8), csz)

    bufs = (bounce_a.at[pl.ds(0, csz)], bounce_b.at[pl.ds(0, csz)])
    pltpu.async_copy(field.at[sl_of(0)], bufs[0], sems[0])
    for j in range(nst):
        buf, sem = bufs[j % 2], sems[j % 2]
        pltpu.make_async_copy(dummy, buf, sem).wait()
        if j + 1 < nst:
            pltpu.async_copy(field.at[sl_of(j + 1)], bufs[(j + 1) % 2],
                             sems[(j + 1) % 2])
        pltpu.sync_copy(buf, fsp.at[sl_of(j)])


def _stage_b_body(gxy, lan, gxn, gyn,
                  fsp, lan_a, lan_b, g_a, g_b, ox_a, ox_b, oy_a, oy_b,
                  semg_a, semg_b, semlan_a, semlan_b):
    s = lax.axis_index("s")
    w = _wid()

    _stage_pipelined_tbl(gxy, fsp, g_a, g_b, (semlan_a, semlan_b),
                         s * (N_LINKS // NS))
    plsc.subcore_barrier()
    iota16 = lax.iota(_i32, 16) * 16
    nib = (NCHB + NW - 1) // NW
    sets = ((lan_a, g_a, ox_a, oy_a, semg_a, semlan_a),
            (lan_b, g_b, ox_b, oy_b, semg_b, semlan_b))

    def chv(i):
        ch = w + i * NW
        return ch, jnp.where(ch < NCHB, ch, 0)

    def lan_fire(i, p):
        _, safe = chv(i)
        lan_v, _, _, _, _, semlan = sets[p]
        off = pl.ds(pl.multiple_of(safe * LANC, 8), LANC)
        pltpu.async_copy(lan.at[off], lan_v, semlan)

    def gath(p):
        lan_v, g_v, _, _, semg, semlan = sets[p]
        pltpu.make_async_copy(lan.at[pl.ds(0, LANC)], lan_v, semlan).wait()
        pltpu.async_copy(fsp.at[lan_v], g_v, semg)

    def red(i, p):
        ch, _ = chv(i)
        _, g_v, ox_v, oy_v, semg, _ = sets[p]
        pltpu.make_async_copy(lan.at[pl.ds(0, LANC)], g_v, semg).wait()

        def body(ii, carry2):
            b = ii * (16 * LPN)
            accx = jnp.zeros((16,), _f32)
            accy = jnp.zeros((16,), _f32)
            for k in range(LPN):
                wv = plsc.load_gather(g_v, [iota16 + (b + k)])
                accx = accx + lax.bitcast_convert_type(
                    wv & jnp.int32(-65536), _f32)
                accy = accy + lax.bitcast_convert_type(
                    lax.shift_left(wv, 16), _f32)
            vs = pl.ds(ii * 16, 16)
            ox_v[vs] = accx * (1.0 / LPN)
            oy_v[vs] = accy * (1.0 / LPN)
            return carry2

        lax.fori_loop(0, CNB // 16, body, 0)

        @pl.when(ch < NCHB)
        def _():
            noff = pl.ds(pl.multiple_of(ch * CNB, 8), CNB)
            pltpu.sync_copy(ox_v, gxn.at[noff])
            pltpu.sync_copy(oy_v, gyn.at[noff])

    lan_fire(0, 0)
    gath(0)
    lan_fire(1, 1)

    def pipe(k, carry):
        c0 = k * 2
        gath(1)
        red(c0, 0)
        lan_fire(c0 + 2, 0)
        gath(0)
        red(c0 + 1, 1)
        lan_fire(c0 + 3, 1)
        return carry

    lax.fori_loop(0, nib // 2 - 1, pipe, 0)
    gath(1)
    red(nib - 2, 0)
    red(nib - 1, 1)


_stage_b = pl.kernel(
    _stage_b_body,
    out_type=tuple(jax.ShapeDtypeStruct((N_NODES,), _f32) for _ in range(2)),
    mesh=_MESH,
    compiler_params=pltpu.CompilerParams(needs_layout_passes=False),
    scratch_types=(
        [pltpu.VMEM_SHARED((N_LINKS,), _i32)]
        + [pltpu.VMEM((LANC,), _i32) for _ in range(4)]
        + [pltpu.VMEM((CNB,), _f32) for _ in range(4)]
        + [pltpu.SemaphoreType.DMA for _ in range(4)]
    ),
)


def _stage_c_body(nx, ny, tr, gxn, gyn, uidx, vel, cen, dif, gox, goy,
                  flux,
                  ps, gps,
                  uidx_a, vel_a, cen_a, dif_a, gox_a, goy_a,
                  up_a, ug_a,
                  uidx_b, vel_b, cen_b, dif_b, gox_b, goy_b,
                  up_b, ug_b,
                  flux_a, flux_b,
                  sx_v, sy_v, st_v, sgx_v, sgy_v, sgp_v,
                  sem_a, sem_b, sst_a, sst_b, slin_a, slin_b):
    s = lax.axis_index("s")

    @pl.when(s < 10)
    def _stage_tables():
        o = pl.multiple_of(s * 10_000, 8)
        slt = pl.ds(o, 10_000)
        pltpu.sync_copy(nx.at[slt], sx_v)
        pltpu.sync_copy(ny.at[slt], sy_v)
        pltpu.sync_copy(tr.at[slt], st_v)
        pltpu.sync_copy(gxn.at[slt], sgx_v)
        pltpu.sync_copy(gyn.at[slt], sgy_v)

        def pbody(j, carry):
            vs = pl.ds(j * 16, 16)
            gxv = sgx_v[vs]
            gyv = sgy_v[vs]
            st_v[vs] = st_v[vs] + sx_v[vs] * gxv + sy_v[vs] * gyv
            gxi = lax.bitcast_convert_type(gxv, _i32) + 0x8000
            gyi = lax.bitcast_convert_type(gyv, _i32) + 0x8000
            sgp_v[vs] = ((gxi & jnp.int32(-65536))
                         | (lax.shift_right_logical(gyi, 16) & 0xFFFF))
            return carry

        lax.fori_loop(0, 10_000 // 16, pbody, 0)
        pltpu.sync_copy(st_v, ps.at[slt])
        pltpu.sync_copy(sgp_v, gps.at[slt])

    plsc.subcore_barrier()
    base = _wid() * LW

    seta = (uidx_a, vel_a, cen_a, dif_a, gox_a, goy_a, up_a, ug_a,
            sem_a, flux_a, sst_a, slin_a)
    setb = (uidx_b, vel_b, cen_b, dif_b, gox_b, goy_b, up_b, ug_b,
            sem_b, flux_b, sst_b, slin_b)

    def load_fire(ci, st):
        (uidx_v, vel_v, cen_v, dif_v, gox_v, goy_v, up_v, ug_v,
         sem) = st[:9]
        slin = st[11]
        sl = pl.ds(pl.multiple_of(base + ci * CL, 8), CL)
        pltpu.sync_copy(uidx.at[sl], uidx_v)
        pltpu.async_copy(vel.at[sl], vel_v, slin)
        pltpu.async_copy(cen.at[sl], cen_v, slin)
        pltpu.async_copy(dif.at[sl], dif_v, slin)
        pltpu.async_copy(gox.at[sl], gox_v, slin)
        pltpu.async_copy(goy.at[sl], goy_v, slin)
        pltpu.async_copy(ps.at[uidx_v], up_v, sem)
        pltpu.async_copy(gps.at[uidx_v], ug_v, sem)

    def drain(st):
        sem = st[8]
        slin = st[11]
        dummy = nx.at[pl.ds(0, CL)]
        for dst in st[1:6]:
            pltpu.make_async_copy(dummy, dst, slin).wait()
        for dst in st[6:8]:
            pltpu.make_async_copy(dummy, dst, sem).wait()

    def compute_store(ci, st):
        _, vel_v, cen_v, dif_v, gox_v, goy_v, up_v, ug_v = st[:8]
        flux_v, sst = st[9:11]
        sl = pl.ds(pl.multiple_of(base + ci * CL, 8), CL)

        @pl.when(ci >= 2)
        def _():
            pltpu.make_async_copy(nx.at[pl.ds(0, CL)], flux_v, sst).wait()

        def vec(j, carry2):
            vs = pl.ds(j * 16, 16)
            wv = ug_v[vs]
            ugx = lax.bitcast_convert_type(wv & jnp.int32(-65536), _f32)
            ugy = lax.bitcast_convert_type(lax.shift_left(wv, 16), _f32)
            up = (up_v[vs] - gox_v[vs] * ugx - goy_v[vs] * ugy)
            ce = cen_v[vs]
            di = dif_v[vs]
            nz = di != 0.0
            den = jnp.where(nz, di, 1.0)
            r = jnp.where(nz, (ce - up) / den, 0.0)
            ar = jnp.abs(r)
            phi = (r + ar) / (1.0 + ar)
            flux_v[vs] = vel_v[vs] * (ce + 0.5 * phi * di)
            return carry2

        lax.fori_loop(0, CL // 16, vec, 0)
        pltpu.async_copy(flux_v, flux.at[sl], sst)

    load_fire(0, seta)

    def pipe(i, carry):
        c0 = i * 2
        load_fire(c0 + 1, setb)
        drain(seta)
        compute_store(c0, seta)
        load_fire(c0 + 2, seta)
        drain(setb)
        compute_store(c0 + 1, setb)
        return carry

    lax.fori_loop(0, NLC // 2, pipe, 0)
    drain(seta)
    compute_store(NLC - 1, seta)
    pltpu.make_async_copy(nx.at[pl.ds(0, CL)], flux_a, sst_a).wait()
    pltpu.make_async_copy(nx.at[pl.ds(0, CL)], flux_b, sst_b).wait()


_stage_c = pl.kernel(
    _stage_c_body,
    out_type=jax.ShapeDtypeStruct((N_LINKS,), _f32),
    mesh=_MESH,
    compiler_params=pltpu.CompilerParams(needs_layout_passes=False),
    scratch_types=(
        [pltpu.VMEM_SHARED((N_NODES,), _f32), pltpu.VMEM_SHARED((N_NODES,), _i32)]
        + 2 * ([pltpu.VMEM((CL,), _i32)]
               + [pltpu.VMEM((CL,), _f32) for _ in range(6)]
               + [pltpu.VMEM((CL,), _i32)])
        + [pltpu.VMEM((CL,), _f32) for _ in range(2)]
        + [pltpu.VMEM((10_000,), _f32) for _ in range(5)]
        + [pltpu.VMEM((10_000,), _i32)]
        + [pltpu.SemaphoreType.DMA for _ in range(6)]
    ),
)


def _stage_d_body(flux, lan, tr, area, dt16, out,
                  fsp, lan_a, lan_b, g_a, g_b, tr_a, tr_b, ar_a, ar_b,
                  o_a, o_b, dt_v,
                  semg_a, semg_b, semlan_a, semlan_b):
    s = lax.axis_index("s")
    w = _wid()

    _stage_pipelined_tbl(flux, fsp, g_a, g_b, (semlan_a, semlan_b),
                         s * (N_LINKS // NS))
    pltpu.sync_copy(dt16, dt_v)
    plsc.subcore_barrier()
    dtv = dt_v[...]
    iota16 = lax.iota(_i32, 16) * 16
    nid = (NCHB + NW - 1) // NW
    sets = ((lan_a, g_a, tr_a, ar_a, o_a, semg_a, semlan_a),
            (lan_b, g_b, tr_b, ar_b, o_b, semg_b, semlan_b))

    def chv(i):
        ch = w + i * NW
        return ch, jnp.where(ch < NCHB, ch, 0)

    def lan_fire(i, p):
        _, safe = chv(i)
        lan_v, _, tr_v, ar_v, _, _, semlan = sets[p]
        off = pl.ds(pl.multiple_of(safe * LANC, 8), LANC)
        noff = pl.ds(pl.multiple_of(safe * CNB, 8), CNB)
        pltpu.async_copy(lan.at[off], lan_v, semlan)
        pltpu.async_copy(tr.at[noff], tr_v, semlan)
        pltpu.async_copy(area.at[noff], ar_v, semlan)

    def gath(p):
        lan_v, g_v, tr_v, ar_v, _, semg, semlan = sets[p]
        pltpu.make_async_copy(lan.at[pl.ds(0, LANC)], lan_v, semlan).wait()
        pltpu.make_async_copy(tr.at[pl.ds(0, CNB)], tr_v, semlan).wait()
        pltpu.make_async_copy(tr.at[pl.ds(0, CNB)], ar_v, semlan).wait()
        pltpu.async_copy(fsp.at[lan_v], g_v, semg)

    def red(i, p):
        ch, _ = chv(i)
        _, g_v, tr_v, ar_v, o_v, semg, _ = sets[p]
        pltpu.make_async_copy(flux.at[pl.ds(0, LANC)], g_v, semg).wait()

        def body(ii, carry2):
            b = ii * (16 * LPN)
            acc = jnp.zeros((16,), _f32)
            for k in range(LPN):
                acc = acc + plsc.load_gather(g_v, [iota16 + (b + k)])
            vs = pl.ds(ii * 16, 16)
            a = ar_v[vs]
            nz = a != 0.0
            asafe = jnp.where(nz, a, 1.0)
            div = jnp.where(nz, acc / asafe, 0.0)
            o_v[vs] = tr_v[vs] + dtv * div
            return carry2

        lax.fori_loop(0, CNB // 16, body, 0)

        @pl.when(ch < NCHB)
        def _():
            pltpu.sync_copy(
                o_v, out.at[pl.ds(pl.multiple_of(ch * CNB, 8), CNB)])

    lan_fire(0, 0)
    gath(0)
    lan_fire(1, 1)

    def pipe(k, carry):
        c0 = k * 2
        gath(1)
        red(c0, 0)
        lan_fire(c0 + 2, 0)
        gath(0)
        red(c0 + 1, 1)
        lan_fire(c0 + 3, 1)
        return carry

    lax.fori_loop(0, nid // 2 - 1, pipe, 0)
    gath(1)
    red(nid - 2, 0)
    red(nid - 1, 1)


_stage_d = pl.kernel(
    _stage_d_body,
    out_type=jax.ShapeDtypeStruct((N_NODES,), _f32),
    mesh=_MESH,
    compiler_params=pltpu.CompilerParams(needs_layout_passes=False),
    scratch_types=(
        [pltpu.VMEM_SHARED((N_LINKS,), _f32)]
        + [pltpu.VMEM((LANC,), _i32) for _ in range(2)]
        + [pltpu.VMEM((LANC,), _f32) for _ in range(2)]
        + [pltpu.VMEM((CNB,), _f32) for _ in range(6)]
        + [pltpu.VMEM((16,), _f32)]
        + [pltpu.SemaphoreType.DMA for _ in range(4)]
    ),
)


def kernel(velocity, tracer, node_x, node_y, cell_area_at_node, dt,
           node_at_link_head, node_at_link_tail, links_at_node,
           upwind_real_idx):
    hidx = node_at_link_head.astype(_i32)
    tidx = node_at_link_tail.astype(_i32)
    uidx = upwind_real_idx.astype(_i32)
    lan_flat = links_at_node.astype(_i32).reshape(-1)
    dt16 = jnp.broadcast_to(dt.astype(_f32), (16,))

    gxy, cen, dif, gox, goy = _stage_a(
        node_x, node_y, tracer, hidx, tidx, velocity)
    gxn, gyn = _stage_b(gxy, lan_flat)
    flux = _stage_c(node_x, node_y, tracer, gxn, gyn, uidx, velocity,
                    cen, dif, gox, goy)
    return _stage_d(flux, lan_flat, tracer, cell_area_at_node, dt16)
